# Initial kernel scaffold; baseline (speedup 1.0000x reference)
import jax
import jax.numpy as jnp
from jax.experimental import pallas as pl


def kernel(x, edge_index, W1, a_src1, a_dst1, b1, W2, a_src2, a_dst2, b2, W3, a_src3, a_dst3, b3):
    raise NotImplementedError("write your pallas kernel here")


# speedup vs baseline: 25.6188x; 25.6188x over previous
import functools

import jax
import jax.numpy as jnp
from jax import lax
from jax.experimental import pallas as pl
from jax.experimental.pallas import tpu as pltpu
from jax.experimental.pallas import tpu_sc as plsc

N = 10000
NPAD = 10240
E = 320000
EPAD = 327680
NC, NS = 2, 16
ROWS_PER_TILE = NPAD // NS
CH = 128

f32 = jnp.float32


def _matmul_body(x_ref, w_ref, as_ref, ad_ref, h_ref, esed_ref, *, split):
  h = jnp.dot(x_ref[...], w_ref[...], preferred_element_type=f32)
  if split:
    for q in range(8):
      h_ref[q] = h[:, q * 32:(q + 1) * 32]
  else:
    h_ref[...] = h
  es = jnp.sum(h * as_ref[0:1, :], axis=1, keepdims=True)
  ed = jnp.sum(h * ad_ref[0:1, :], axis=1, keepdims=True)
  z = jnp.zeros((h.shape[0], 6), f32)
  esed_ref[...] = jnp.concatenate([es, ed, z], axis=1)


def _tc_matmul(x, w, a_s, a_d, *, split):
  bn = 1024
  g = NPAD // bn
  din, dout = w.shape
  if split:
    h_shape = jax.ShapeDtypeStruct((8, NPAD, 32), f32)
    h_spec = pl.BlockSpec((8, bn, 32), lambda i: (0, i, 0))
  else:
    h_shape = jax.ShapeDtypeStruct((NPAD, dout), f32)
    h_spec = pl.BlockSpec((bn, dout), lambda i: (i, 0))
  return pl.pallas_call(
      functools.partial(_matmul_body, split=split),
      grid=(g,),
      in_specs=[
          pl.BlockSpec((bn, din), lambda i: (i, 0)),
          pl.BlockSpec((din, dout), lambda i: (0, 0)),
          pl.BlockSpec((8, dout), lambda i: (0, 0)),
          pl.BlockSpec((8, dout), lambda i: (0, 0)),
      ],
      out_specs=[h_spec, pl.BlockSpec((bn, 8), lambda i: (i, 0))],
      out_shape=[h_shape, jax.ShapeDtypeStruct((NPAD, 8), f32)],
  )(x, w, a_s, a_d)


def _norm_matmul_body(u0_ref, u1_ref, d0_ref, d1_ref, b_ref, w_ref, as_ref,
                      ad_ref, h_ref, esed_ref):
  den = d0_ref[...] + d1_ref[...]
  den = jnp.where(den > 0, den, 1.0)
  xin = (u0_ref[...] + u1_ref[...]) / den + b_ref[0:1, :]
  xin = jnp.where(xin > 0, xin, 0.01 * xin)
  h = jnp.dot(xin, w_ref[...], preferred_element_type=f32)
  es = jnp.sum(h * as_ref[0:1, :], axis=1, keepdims=True)
  ed = jnp.sum(h * ad_ref[0:1, :], axis=1, keepdims=True)
  z = jnp.zeros((h.shape[0], 6), f32)
  esed_ref[...] = jnp.concatenate([es, ed, z], axis=1)
  h_ref[...] = h


def _tc_norm_matmul(u0, u1, d0, d1, b, w, a_s, a_d):
  bn = 1024
  g = NPAD // bn
  din, dout = w.shape
  return pl.pallas_call(
      _norm_matmul_body,
      grid=(g,),
      in_specs=[
          pl.BlockSpec((bn, din), lambda i: (i, 0)),
          pl.BlockSpec((bn, din), lambda i: (i, 0)),
          pl.BlockSpec((bn, 1), lambda i: (i, 0)),
          pl.BlockSpec((bn, 1), lambda i: (i, 0)),
          pl.BlockSpec((8, din), lambda i: (0, 0)),
          pl.BlockSpec((din, dout), lambda i: (0, 0)),
          pl.BlockSpec((8, dout), lambda i: (0, 0)),
          pl.BlockSpec((8, dout), lambda i: (0, 0)),
      ],
      out_specs=[
          pl.BlockSpec((bn, dout), lambda i: (i, 0)),
          pl.BlockSpec((bn, 8), lambda i: (i, 0)),
      ],
      out_shape=[
          jax.ShapeDtypeStruct((NPAD, dout), f32),
          jax.ShapeDtypeStruct((NPAD, 8), f32),
      ],
  )(u0, u1, d0, d1, b, w, a_s, a_d)


def _final_body(u0_ref, u1_ref, d0_ref, d1_ref, b_ref, out_ref):
  den = d0_ref[...] + d1_ref[...]
  den = jnp.where(den > 0, den, 1.0)
  out_ref[...] = (u0_ref[...] + u1_ref[...])[:, :40] / den + b_ref[0:1, :40]


def _tc_final(u0, u1, d0, d1, b):
  bn = 1000
  return pl.pallas_call(
      _final_body,
      grid=(10,),
      in_specs=[
          pl.BlockSpec((bn, 48), lambda i: (i, 0)),
          pl.BlockSpec((bn, 48), lambda i: (i, 0)),
          pl.BlockSpec((bn, 1), lambda i: (i, 0)),
          pl.BlockSpec((bn, 1), lambda i: (i, 0)),
          pl.BlockSpec((8, 48), lambda i: (0, 0)),
      ],
      out_specs=pl.BlockSpec((bn, 40), lambda i: (i, 0)),
      out_shape=jax.ShapeDtypeStruct((N, 40), f32),
  )(u0, u1, d0, d1, b)


def _sc_edge_body(refs, *, d2, ngrp, nch, normalize):
  if normalize:
    (h_hbm, srcm, dstm, esedt, b_hbm, x2_hbm,
     es_v, ed_v, sidx2, didx2, gidx, rows, exall, exb, b_v,
     u_sh, den_sh, sem0, sem1) = refs
  else:
    (h_hbm, srcm, dstm, esedt, u_hbm, den_hbm,
     es_v, ed_v, sidx2, didx2, gidx, rows, exall, exb,
     u_sh, den_sh, sem0, sem1) = refs
  sems = [sem0, sem1]
  c = lax.axis_index("c")
  s = lax.axis_index("s")
  nvec = d2 // 16
  feature_split = ngrp > 1

  chunk_base = s * nch if feature_split else (c * NS + s) * nch
  pltpu.sync_copy(srcm.at[pl.ds(chunk_base, nch)], sidx2)
  pltpu.sync_copy(dstm.at[pl.ds(chunk_base, nch)], didx2)
  pltpu.sync_copy(esedt.at[0], es_v)
  pltpu.sync_copy(esedt.at[1], ed_v)

  def _lanemax(v_ref):
    def mxi(i, cur):
      return jnp.maximum(cur, v_ref[pl.ds(i * 16, 16)])
    return lax.fori_loop(1, NPAD // 16, mxi, v_ref[pl.ds(0, 16)])

  iota16 = lax.iota(jnp.int32, 16)
  def _bfly(m):
    for step in (8, 4, 2, 1):
      exb[pl.ds(0, 16)] = m
      m = jnp.maximum(m, plsc.load_gather(exb, [jnp.bitwise_xor(iota16, step)]))
    return m
  mbar = jnp.maximum(_bfly(_lanemax(es_v)) + _bfly(_lanemax(ed_v)),
                     jnp.zeros((16,), f32))

  zv = jnp.zeros((16,), f32)
  r0 = s * ROWS_PER_TILE

  def _issue(jj, p, goff):
    def gi(i, _):
      sl = pl.ds(i * 16, 16)
      gidx[p, sl] = sidx2[jj, sl] + goff
      return 0
    lax.fori_loop(0, CH // 16, gi, 0)
    pltpu.async_copy(h_hbm.at[gidx.at[p]], rows.at[p], sems[p])

  for g in range(ngrp):
    grp = c * ngrp + g
    goff = grp * NPAD if feature_split else 0

    def _zrows(i, _):
      for bb in range(2):
        for v in range(nvec):
          rows[bb, i, pl.ds(v * 16, 16)] = zv
      return 0
    lax.fori_loop(0, CH, _zrows, 0)
    def _zexb(i, _):
      exb[pl.ds(i * 16, 16)] = zv
      return 0
    lax.fori_loop(0, CH // 16, _zexb, 0)
    for bi in range(ROWS_PER_TILE // CH):
      pltpu.sync_copy(rows.at[0], u_sh.at[pl.ds(r0 + bi * CH, CH), :])
      if g == 0:
        pltpu.sync_copy(exb, den_sh.at[pl.ds(r0 + bi * CH, CH)])
    plsc.subcore_barrier()

    _issue(0, 0, goff)

    def chunk_body(j, _):
      p = j % 2
      @pl.when(j + 1 < nch)
      def _():
        @pl.when(p == 1)
        def _():
          _issue(j + 1, 0, goff)
        @pl.when(p == 0)
        def _():
          _issue(j + 1, 1, goff)

      if g == 0:
        def exi(i, _):
          sl = pl.ds(i * 16, 16)
          e = (plsc.load_gather(es_v, [sidx2[j, sl]]) +
               plsc.load_gather(ed_v, [didx2[j, sl]]))
          e = jnp.where(e > 0, e, 0.2 * e)
          exall[j, sl] = jnp.exp(e - mbar)
          return 0
        lax.fori_loop(0, CH // 16, exi, 0)

      @pl.when(p == 0)
      def _():
        pltpu.make_async_copy(h_hbm.at[gidx.at[0]], rows.at[0], sems[0]).wait()
      @pl.when(p == 1)
      def _():
        pltpu.make_async_copy(h_hbm.at[gidx.at[1]], rows.at[1], sems[1]).wait()

      bj = jnp.broadcast_to(j, (16,)).astype(jnp.int32)
      def scale4(q, _):
        for kk in range(4):
          k = q * 4 + kk
          bk = jnp.broadcast_to(k, (16,)).astype(jnp.int32)
          sv = plsc.load_gather(exall, [bj, bk])
          for v in range(nvec):
            sl = pl.ds(v * 16, 16)
            rows[p, k, sl] = rows[p, k, sl] * sv
        return 0
      lax.fori_loop(0, CH // 4, scale4, 0)

      pltpu.sync_copy(rows.at[p], u_sh.at[didx2.at[j]], add=True)
      if g == 0:
        pltpu.sync_copy(exall.at[j], den_sh.at[didx2.at[j]], add=True)
      return 0

    lax.fori_loop(0, nch, chunk_body, 0)
    plsc.subcore_barrier()

    if normalize:
      pltpu.sync_copy(b_hbm.at[0, pl.ds(grp * d2, d2)], b_v)
      bvecs = [b_v[pl.ds(v * 16, 16)] for v in range(nvec)]
    for bi in range(ROWS_PER_TILE // CH):
      rsl = pl.ds(r0 + bi * CH, CH)
      pltpu.sync_copy(u_sh.at[rsl, :], rows.at[0])
      pltpu.sync_copy(den_sh.at[rsl], exb)
      if normalize:
        def rdi(i, _):
          sl = pl.ds(i * 16, 16)
          d = exb[sl]
          exb[sl] = 1.0 / jnp.where(d > 0, d, 1.0)
          return 0
        lax.fori_loop(0, CH // 16, rdi, 0)

        def nrow4(q, _):
          for kk in range(4):
            k = q * 4 + kk
            bk = jnp.broadcast_to(k, (16,)).astype(jnp.int32)
            rv = plsc.load_gather(exb, [bk])
            for v in range(nvec):
              sl = pl.ds(v * 16, 16)
              t = rows[0, k, sl] * rv + bvecs[v]
              rows[0, k, sl] = jnp.where(t > 0, t, 0.01 * t)
          return 0
        lax.fori_loop(0, CH // 4, nrow4, 0)
        pltpu.sync_copy(rows.at[0], x2_hbm.at[grp, rsl, :])
      else:
        pltpu.sync_copy(rows.at[0], u_hbm.at[c, rsl, :])
        pltpu.sync_copy(exb, den_hbm.at[c, rsl])


def _sc_layer(h, srcm, dstm, esedt, b=None, *, d2, ngrp, normalize):
  feature_split = ngrp > 1
  nch = (EPAD // CH) // (NS if feature_split else (NS * NC))
  mesh = plsc.VectorSubcoreMesh(core_axis_name="c", subcore_axis_name="s",
                                num_cores=NC, num_subcores=NS)
  if normalize:
    out_type = jax.ShapeDtypeStruct((NC * ngrp, NPAD, d2), f32)
  else:
    out_type = (jax.ShapeDtypeStruct((2, NPAD, d2), f32),
                jax.ShapeDtypeStruct((2, NPAD), f32))
  scratch = [
      pltpu.VMEM((NPAD,), f32),
      pltpu.VMEM((NPAD,), f32),
      pltpu.VMEM((nch, CH), jnp.int32),
      pltpu.VMEM((nch, CH), jnp.int32),
      pltpu.VMEM((2, CH), jnp.int32),
      pltpu.VMEM((2, CH, d2), f32),
      pltpu.VMEM((nch, CH), f32),
      pltpu.VMEM((CH,), f32),
  ]
  if normalize:
    scratch.append(pltpu.VMEM((d2,), f32))
  scratch += [
      pltpu.VMEM_SHARED((NPAD, d2), f32),
      pltpu.VMEM_SHARED((NPAD,), f32),
      pltpu.SemaphoreType.DMA,
      pltpu.SemaphoreType.DMA,
  ]

  def body(*refs):
    _sc_edge_body(refs, d2=d2, ngrp=ngrp, nch=nch, normalize=normalize)

  cp = pltpu.CompilerParams(
      needs_layout_passes=False,
      use_tc_tiling_on_sc=False)
  fn = pl.kernel(body, out_type=out_type, mesh=mesh, scratch_types=scratch,
                 compiler_params=cp)
  if normalize:
    return fn(h, srcm, dstm, esedt, b)
  return fn(h, srcm, dstm, esedt)


def kernel(x, edge_index, W1, a_src1, a_dst1, b1, W2, a_src2, a_dst2, b2,
           W3, a_src3, a_dst3, b3):
  src = edge_index[0]
  dst = edge_index[1]
  npad_e = EPAD - E
  padidx = (jnp.arange(npad_e, dtype=jnp.int32) % (NPAD - N)) + N
  srcm = jnp.concatenate([src, padidx]).reshape(EPAD // CH, CH)
  dstm = jnp.concatenate([dst, padidx]).reshape(EPAD // CH, CH)
  xp = jnp.pad(x, ((0, NPAD - N), (0, 0)))

  bc8 = lambda v: jnp.broadcast_to(v[None, :], (8, v.shape[0]))
  w3p = jnp.pad(W3, ((0, 0), (0, 8)))
  a_src3p = jnp.pad(a_src3, (0, 8))
  a_dst3p = jnp.pad(a_dst3, (0, 8))
  b3p = jnp.pad(b3, (0, 8))

  h1, esed1 = _tc_matmul(xp, W1, bc8(a_src1), bc8(a_dst1), split=True)
  x2_8d = _sc_layer(h1.reshape(8 * NPAD, 32), srcm, dstm, esed1.T,
                    bc8(b1), d2=32, ngrp=4, normalize=True)
  x2 = x2_8d.transpose(1, 0, 2).reshape(NPAD, 256)

  h2, esed2 = _tc_matmul(x2, W2, bc8(a_src2), bc8(a_dst2), split=False)
  u2, den2 = _sc_layer(h2, srcm, dstm, esed2.T,
                       d2=32, ngrp=1, normalize=False)

  h3, esed3 = _tc_norm_matmul(u2[0], u2[1], den2[0][:, None], den2[1][:, None],
                              bc8(b2), w3p, bc8(a_src3p), bc8(a_dst3p))
  u3, den3 = _sc_layer(h3, srcm, dstm, esed3.T,
                       d2=48, ngrp=1, normalize=False)

  return _tc_final(u3[0], u3[1], den3[0][:, None], den3[1][:, None], bc8(b3p))


# speedup vs baseline: 31.4277x; 1.2267x over previous
import functools

import jax
import jax.numpy as jnp
from jax import lax
from jax.experimental import pallas as pl
from jax.experimental.pallas import tpu as pltpu
from jax.experimental.pallas import tpu_sc as plsc

N = 10000
NPAD = 10240
E = 320000
EPAD = 327680
NC, NS = 2, 16
ROWS_PER_TILE = NPAD // NS
CH = 128
NBUF = 4

f32 = jnp.float32


def _matmul_body(x_ref, w_ref, as_ref, ad_ref, h_ref, esed_ref, *, split):
  h = jnp.dot(x_ref[...], w_ref[...], preferred_element_type=f32)
  if split:
    for q in range(8):
      h_ref[q] = h[:, q * 32:(q + 1) * 32]
  else:
    h_ref[...] = h
  es = jnp.sum(h * as_ref[0:1, :], axis=1, keepdims=True)
  ed = jnp.sum(h * ad_ref[0:1, :], axis=1, keepdims=True)
  z = jnp.zeros((h.shape[0], 6), f32)
  esed_ref[...] = jnp.concatenate([es, ed, z], axis=1)


def _tc_matmul(x, w, a_s, a_d, *, split):
  bn = 1024
  g = NPAD // bn
  din, dout = w.shape
  if split:
    h_shape = jax.ShapeDtypeStruct((8, NPAD, 32), f32)
    h_spec = pl.BlockSpec((8, bn, 32), lambda i: (0, i, 0))
  else:
    h_shape = jax.ShapeDtypeStruct((NPAD, dout), f32)
    h_spec = pl.BlockSpec((bn, dout), lambda i: (i, 0))
  return pl.pallas_call(
      functools.partial(_matmul_body, split=split),
      grid=(g,),
      in_specs=[
          pl.BlockSpec((bn, din), lambda i: (i, 0)),
          pl.BlockSpec((din, dout), lambda i: (0, 0)),
          pl.BlockSpec((8, dout), lambda i: (0, 0)),
          pl.BlockSpec((8, dout), lambda i: (0, 0)),
      ],
      out_specs=[h_spec, pl.BlockSpec((bn, 8), lambda i: (i, 0))],
      out_shape=[h_shape, jax.ShapeDtypeStruct((NPAD, 8), f32)],
  )(x, w, a_s, a_d)


def _matmul_split_in_body(x_ref, w_ref, as_ref, ad_ref, h_ref, esed_ref):
  acc = jnp.dot(x_ref[0], w_ref[0], preferred_element_type=f32)
  for q in range(1, 8):
    acc += jnp.dot(x_ref[q], w_ref[q], preferred_element_type=f32)
  es = jnp.sum(acc * as_ref[0:1, :], axis=1, keepdims=True)
  ed = jnp.sum(acc * ad_ref[0:1, :], axis=1, keepdims=True)
  z = jnp.zeros((acc.shape[0], 6), f32)
  esed_ref[...] = jnp.concatenate([es, ed, z], axis=1)
  h_ref[...] = acc


def _tc_matmul_split_in(x8, w, a_s, a_d):
  bn = 1024
  g = NPAD // bn
  dout = w.shape[1]
  w8 = w.reshape(8, 32, dout)
  return pl.pallas_call(
      _matmul_split_in_body,
      grid=(g,),
      in_specs=[
          pl.BlockSpec((8, bn, 32), lambda i: (0, i, 0)),
          pl.BlockSpec((8, 32, dout), lambda i: (0, 0, 0)),
          pl.BlockSpec((8, dout), lambda i: (0, 0)),
          pl.BlockSpec((8, dout), lambda i: (0, 0)),
      ],
      out_specs=[
          pl.BlockSpec((bn, dout), lambda i: (i, 0)),
          pl.BlockSpec((bn, 8), lambda i: (i, 0)),
      ],
      out_shape=[
          jax.ShapeDtypeStruct((NPAD, dout), f32),
          jax.ShapeDtypeStruct((NPAD, 8), f32),
      ],
  )(x8, w8, a_s, a_d)


def _norm_matmul_body(u0_ref, u1_ref, d0_ref, d1_ref, b_ref, w_ref, as_ref,
                      ad_ref, h_ref, esed_ref):
  den = d0_ref[...] + d1_ref[...]
  den = jnp.where(den > 0, den, 1.0)
  xin = (u0_ref[...] + u1_ref[...]) / den + b_ref[0:1, :]
  xin = jnp.where(xin > 0, xin, 0.01 * xin)
  h = jnp.dot(xin, w_ref[...], preferred_element_type=f32)
  es = jnp.sum(h * as_ref[0:1, :], axis=1, keepdims=True)
  ed = jnp.sum(h * ad_ref[0:1, :], axis=1, keepdims=True)
  z = jnp.zeros((h.shape[0], 6), f32)
  esed_ref[...] = jnp.concatenate([es, ed, z], axis=1)
  h_ref[...] = h


def _tc_norm_matmul(u0, u1, d0, d1, b, w, a_s, a_d):
  bn = 1024
  g = NPAD // bn
  din, dout = w.shape
  return pl.pallas_call(
      _norm_matmul_body,
      grid=(g,),
      in_specs=[
          pl.BlockSpec((bn, din), lambda i: (i, 0)),
          pl.BlockSpec((bn, din), lambda i: (i, 0)),
          pl.BlockSpec((bn, 1), lambda i: (i, 0)),
          pl.BlockSpec((bn, 1), lambda i: (i, 0)),
          pl.BlockSpec((8, din), lambda i: (0, 0)),
          pl.BlockSpec((din, dout), lambda i: (0, 0)),
          pl.BlockSpec((8, dout), lambda i: (0, 0)),
          pl.BlockSpec((8, dout), lambda i: (0, 0)),
      ],
      out_specs=[
          pl.BlockSpec((bn, dout), lambda i: (i, 0)),
          pl.BlockSpec((bn, 8), lambda i: (i, 0)),
      ],
      out_shape=[
          jax.ShapeDtypeStruct((NPAD, dout), f32),
          jax.ShapeDtypeStruct((NPAD, 8), f32),
      ],
  )(u0, u1, d0, d1, b, w, a_s, a_d)


def _final_body(u0_ref, u1_ref, d0_ref, d1_ref, b_ref, out_ref):
  den = d0_ref[...] + d1_ref[...]
  den = jnp.where(den > 0, den, 1.0)
  out_ref[...] = (u0_ref[...] + u1_ref[...])[:, :40] / den + b_ref[0:1, :40]


def _tc_final(u0, u1, d0, d1, b):
  bn = 1000
  return pl.pallas_call(
      _final_body,
      grid=(10,),
      in_specs=[
          pl.BlockSpec((bn, 48), lambda i: (i, 0)),
          pl.BlockSpec((bn, 48), lambda i: (i, 0)),
          pl.BlockSpec((bn, 1), lambda i: (i, 0)),
          pl.BlockSpec((bn, 1), lambda i: (i, 0)),
          pl.BlockSpec((8, 48), lambda i: (0, 0)),
      ],
      out_specs=pl.BlockSpec((bn, 40), lambda i: (i, 0)),
      out_shape=jax.ShapeDtypeStruct((N, 40), f32),
  )(u0, u1, d0, d1, b)


def _sc_edge_body(refs, *, d2, ngrp, nch, normalize):
  if normalize:
    (h_hbm, srcm, dstm, esedt, b_hbm, x2_hbm,
     es_v, ed_v, sidx2, didx2, gidx, rows, exall, exb, b_v,
     u_sh, den_sh, *sems) = refs
  else:
    (h_hbm, srcm, dstm, esedt, u_hbm, den_hbm,
     es_v, ed_v, sidx2, didx2, gidx, rows, exall, exb,
     u_sh, den_sh, *sems) = refs
  semg = sems[0:NBUF]
  semsc = sems[NBUF:2 * NBUF]
  semd = sems[2 * NBUF:2 * NBUF + 2]
  c = lax.axis_index("c")
  s = lax.axis_index("s")
  nvec = d2 // 16
  feature_split = ngrp > 1

  chunk_base = s * nch if feature_split else (c * NS + s) * nch
  pltpu.sync_copy(srcm.at[pl.ds(chunk_base, nch)], sidx2)
  pltpu.sync_copy(dstm.at[pl.ds(chunk_base, nch)], didx2)
  pltpu.sync_copy(esedt.at[0], es_v)
  pltpu.sync_copy(esedt.at[1], ed_v)

  def _lanemax(v_ref):
    def mxi(i, cur):
      return jnp.maximum(cur, v_ref[pl.ds(i * 16, 16)])
    return lax.fori_loop(1, NPAD // 16, mxi, v_ref[pl.ds(0, 16)])

  iota16 = lax.iota(jnp.int32, 16)
  def _bfly(m):
    for step in (8, 4, 2, 1):
      exb[pl.ds(0, 16)] = m
      m = jnp.maximum(m, plsc.load_gather(exb, [jnp.bitwise_xor(iota16, step)]))
    return m
  mbar = jnp.maximum(_bfly(_lanemax(es_v)) + _bfly(_lanemax(ed_v)),
                     jnp.zeros((16,), f32))

  zv = jnp.zeros((16,), f32)
  r0 = s * ROWS_PER_TILE

  def exi(j, _):
    for i in range(CH // 16):
      sl = pl.ds(i * 16, 16)
      e = (plsc.load_gather(es_v, [sidx2[j, sl]]) +
           plsc.load_gather(ed_v, [didx2[j, sl]]))
      e = jnp.where(e > 0, e, 0.2 * e)
      exall[j, sl] = jnp.exp(e - mbar)
    return 0
  lax.fori_loop(0, nch, exi, 0)

  def _issue_gather(jj, b, goff):
    def gi(i, _):
      sl = pl.ds(i * 16, 16)
      gidx[b, sl] = sidx2[jj, sl] + goff
      return 0
    lax.fori_loop(0, CH // 16, gi, 0)
    pltpu.async_copy(h_hbm.at[gidx.at[b]], rows.at[b], semg[b])

  for g in range(ngrp):
    grp = c * ngrp + g
    goff = grp * NPAD if feature_split else 0

    def _zrows(i, _):
      for bb in range(NBUF):
        for v in range(nvec):
          rows[bb, i, pl.ds(v * 16, 16)] = zv
      return 0
    lax.fori_loop(0, CH, _zrows, 0)
    def _zexb(i, _):
      exb[pl.ds(i * 16, 16)] = zv
      return 0
    lax.fori_loop(0, CH // 16, _zexb, 0)
    for bi in range(ROWS_PER_TILE // CH):
      pltpu.sync_copy(rows.at[0], u_sh.at[pl.ds(r0 + bi * CH, CH), :])
      if g == 0:
        pltpu.sync_copy(exb, den_sh.at[pl.ds(r0 + bi * CH, CH)])
    plsc.subcore_barrier()

    _issue_gather(0, 0, goff)
    _issue_gather(1, 1, goff)

    def quad_body(i, _):
      for t in range(NBUF):
        j = i * NBUF + t
        bn2 = (t + 2) % NBUF
        @pl.when(j >= 2)
        def _():
          pltpu.make_async_copy(rows.at[bn2], u_sh.at[didx2.at[j - 2]],
                                semsc[bn2]).wait()
        @pl.when(j + 2 < nch)
        def _():
          _issue_gather(j + 2, bn2, goff)

        pltpu.make_async_copy(h_hbm.at[gidx.at[t]], rows.at[t],
                              semg[t]).wait()
        bj = jnp.broadcast_to(j, (16,)).astype(jnp.int32)
        def scale8(q, _):
          for kk in range(8):
            k = q * 8 + kk
            bk = jnp.broadcast_to(k, (16,)).astype(jnp.int32)
            sv = plsc.load_gather(exall, [bj, bk])
            for v in range(nvec):
              sl = pl.ds(v * 16, 16)
              rows[t, k, sl] = rows[t, k, sl] * sv
          return 0
        lax.fori_loop(0, CH // 8, scale8, 0)

        if g == 0:
          @pl.when(j >= 2)
          def _():
            pltpu.make_async_copy(exall.at[j - 2], den_sh.at[didx2.at[j - 2]],
                                  semd[t % 2]).wait()
        pltpu.async_copy(rows.at[t], u_sh.at[didx2.at[j]], semsc[t], add=True)
        if g == 0:
          pltpu.async_copy(exall.at[j], den_sh.at[didx2.at[j]], semd[t % 2],
                           add=True)
      return 0

    lax.fori_loop(0, nch // NBUF, quad_body, 0)
    for j in (nch - 2, nch - 1):
      bb = j % NBUF
      pltpu.make_async_copy(rows.at[bb], u_sh.at[didx2.at[j]],
                            semsc[bb]).wait()
      if g == 0:
        pltpu.make_async_copy(exall.at[j], den_sh.at[didx2.at[j]],
                              semd[j % 2]).wait()
    plsc.subcore_barrier()

    if normalize:
      pltpu.sync_copy(b_hbm.at[0, pl.ds(grp * d2, d2)], b_v)
      bvecs = [b_v[pl.ds(v * 16, 16)] for v in range(nvec)]
    for bi in range(ROWS_PER_TILE // CH):
      rsl = pl.ds(r0 + bi * CH, CH)
      pltpu.sync_copy(u_sh.at[rsl, :], rows.at[0])
      pltpu.sync_copy(den_sh.at[rsl], exb)
      if normalize:
        def rdi(i, _):
          sl = pl.ds(i * 16, 16)
          d = exb[sl]
          exb[sl] = 1.0 / jnp.where(d > 0, d, 1.0)
          return 0
        lax.fori_loop(0, CH // 16, rdi, 0)

        def nrow4(q, _):
          for kk in range(4):
            k = q * 4 + kk
            bk = jnp.broadcast_to(k, (16,)).astype(jnp.int32)
            rv = plsc.load_gather(exb, [bk])
            for v in range(nvec):
              sl = pl.ds(v * 16, 16)
              t = rows[0, k, sl] * rv + bvecs[v]
              rows[0, k, sl] = jnp.where(t > 0, t, 0.01 * t)
          return 0
        lax.fori_loop(0, CH // 4, nrow4, 0)
        pltpu.sync_copy(rows.at[0], x2_hbm.at[grp, rsl, :])
      else:
        pltpu.sync_copy(rows.at[0], u_hbm.at[c, rsl, :])
        pltpu.sync_copy(exb, den_hbm.at[c, rsl])


def _sc_layer(h, srcm, dstm, esedt, b=None, *, d2, ngrp, normalize):
  feature_split = ngrp > 1
  nch = (EPAD // CH) // (NS if feature_split else (NS * NC))
  mesh = plsc.VectorSubcoreMesh(core_axis_name="c", subcore_axis_name="s",
                                num_cores=NC, num_subcores=NS)
  if normalize:
    out_type = jax.ShapeDtypeStruct((NC * ngrp, NPAD, d2), f32)
  else:
    out_type = (jax.ShapeDtypeStruct((2, NPAD, d2), f32),
                jax.ShapeDtypeStruct((2, NPAD), f32))
  scratch = [
      pltpu.VMEM((NPAD,), f32),
      pltpu.VMEM((NPAD,), f32),
      pltpu.VMEM((nch, CH), jnp.int32),
      pltpu.VMEM((nch, CH), jnp.int32),
      pltpu.VMEM((NBUF, CH), jnp.int32),
      pltpu.VMEM((NBUF, CH, d2), f32),
      pltpu.VMEM((nch, CH), f32),
      pltpu.VMEM((CH,), f32),
  ]
  if normalize:
    scratch.append(pltpu.VMEM((d2,), f32))
  scratch += [
      pltpu.VMEM_SHARED((NPAD, d2), f32),
      pltpu.VMEM_SHARED((NPAD,), f32),
  ] + [pltpu.SemaphoreType.DMA] * (2 * NBUF + 2)

  def body(*refs):
    _sc_edge_body(refs, d2=d2, ngrp=ngrp, nch=nch, normalize=normalize)

  cp = pltpu.CompilerParams(
      needs_layout_passes=False,
      use_tc_tiling_on_sc=False)
  fn = pl.kernel(body, out_type=out_type, mesh=mesh, scratch_types=scratch,
                 compiler_params=cp)
  if normalize:
    return fn(h, srcm, dstm, esedt, b)
  return fn(h, srcm, dstm, esedt)


def kernel(x, edge_index, W1, a_src1, a_dst1, b1, W2, a_src2, a_dst2, b2,
           W3, a_src3, a_dst3, b3):
  src = edge_index[0]
  dst = edge_index[1]
  npad_e = EPAD - E
  padidx = (jnp.arange(npad_e, dtype=jnp.int32) % (NPAD - N)) + N
  srcm = jnp.concatenate([src, padidx]).reshape(EPAD // CH, CH)
  dstm = jnp.concatenate([dst, padidx]).reshape(EPAD // CH, CH)
  xp = jnp.pad(x, ((0, NPAD - N), (0, 0)))

  bc8 = lambda v: jnp.broadcast_to(v[None, :], (8, v.shape[0]))
  w3p = jnp.pad(W3, ((0, 0), (0, 8)))
  a_src3p = jnp.pad(a_src3, (0, 8))
  a_dst3p = jnp.pad(a_dst3, (0, 8))
  b3p = jnp.pad(b3, (0, 8))

  h1, esed1 = _tc_matmul(xp, W1, bc8(a_src1), bc8(a_dst1), split=True)
  x2_8d = _sc_layer(h1.reshape(8 * NPAD, 32), srcm, dstm, esed1.T,
                    bc8(b1), d2=32, ngrp=4, normalize=True)

  h2, esed2 = _tc_matmul_split_in(x2_8d, W2, bc8(a_src2), bc8(a_dst2))
  u2, den2 = _sc_layer(h2, srcm, dstm, esed2.T,
                       d2=32, ngrp=1, normalize=False)

  h3, esed3 = _tc_norm_matmul(u2[0], u2[1], den2[0][:, None], den2[1][:, None],
                              bc8(b2), w3p, bc8(a_src3p), bc8(a_dst3p))
  u3, den3 = _sc_layer(h3, srcm, dstm, esed3.T,
                       d2=48, ngrp=1, normalize=False)

  return _tc_final(u3[0], u3[1], den3[0][:, None], den3[1][:, None], bc8(b3p))


# speedup vs baseline: 49.4668x; 1.5740x over previous
import functools

import jax
import jax.numpy as jnp
from jax import lax
from jax.experimental import pallas as pl
from jax.experimental.pallas import tpu as pltpu
from jax.experimental.pallas import tpu_sc as plsc

N = 10000
NPAD = 10240
E = 320000
EPAD = 327680
NC, NS = 2, 16
ROWS_PER_TILE = NPAD // NS
CH = 128
NBUF = 4

f32 = jnp.float32


def _matmul_body(x_ref, w_ref, as_ref, ad_ref, h_ref, esed_ref, *, split):
  h = jnp.dot(x_ref[...], w_ref[...], preferred_element_type=f32)
  if split:
    for q in range(8):
      h_ref[q] = h[:, q * 32:(q + 1) * 32]
  else:
    h_ref[...] = h
  es = jnp.sum(h * as_ref[0:1, :], axis=1, keepdims=True)
  ed = jnp.sum(h * ad_ref[0:1, :], axis=1, keepdims=True)
  z = jnp.zeros((h.shape[0], 6), f32)
  esed_ref[...] = jnp.concatenate([es, ed, z], axis=1)


def _tc_matmul(x, w, a_s, a_d, *, split):
  bn = 1024
  g = NPAD // bn
  din, dout = w.shape
  if split:
    h_shape = jax.ShapeDtypeStruct((8, NPAD, 32), f32)
    h_spec = pl.BlockSpec((8, bn, 32), lambda i: (0, i, 0))
  else:
    h_shape = jax.ShapeDtypeStruct((NPAD, dout), f32)
    h_spec = pl.BlockSpec((bn, dout), lambda i: (i, 0))
  return pl.pallas_call(
      functools.partial(_matmul_body, split=split),
      grid=(g,),
      in_specs=[
          pl.BlockSpec((bn, din), lambda i: (i, 0)),
          pl.BlockSpec((din, dout), lambda i: (0, 0)),
          pl.BlockSpec((8, dout), lambda i: (0, 0)),
          pl.BlockSpec((8, dout), lambda i: (0, 0)),
      ],
      out_specs=[h_spec, pl.BlockSpec((bn, 8), lambda i: (i, 0))],
      out_shape=[h_shape, jax.ShapeDtypeStruct((NPAD, 8), f32)],
  )(x, w, a_s, a_d)


def _matmul_split_in_body(x_ref, w_ref, as_ref, ad_ref, h_ref, esed_ref):
  acc = jnp.dot(x_ref[0], w_ref[0], preferred_element_type=f32)
  for q in range(1, 8):
    acc += jnp.dot(x_ref[q], w_ref[q], preferred_element_type=f32)
  es = jnp.sum(acc * as_ref[0:1, :], axis=1, keepdims=True)
  ed = jnp.sum(acc * ad_ref[0:1, :], axis=1, keepdims=True)
  z = jnp.zeros((acc.shape[0], 6), f32)
  esed_ref[...] = jnp.concatenate([es, ed, z], axis=1)
  h_ref[...] = acc


def _tc_matmul_split_in(x8, w, a_s, a_d):
  bn = 1024
  g = NPAD // bn
  dout = w.shape[1]
  w8 = w.reshape(8, 32, dout)
  return pl.pallas_call(
      _matmul_split_in_body,
      grid=(g,),
      in_specs=[
          pl.BlockSpec((8, bn, 32), lambda i: (0, i, 0)),
          pl.BlockSpec((8, 32, dout), lambda i: (0, 0, 0)),
          pl.BlockSpec((8, dout), lambda i: (0, 0)),
          pl.BlockSpec((8, dout), lambda i: (0, 0)),
      ],
      out_specs=[
          pl.BlockSpec((bn, dout), lambda i: (i, 0)),
          pl.BlockSpec((bn, 8), lambda i: (i, 0)),
      ],
      out_shape=[
          jax.ShapeDtypeStruct((NPAD, dout), f32),
          jax.ShapeDtypeStruct((NPAD, 8), f32),
      ],
  )(x8, w8, a_s, a_d)


def _norm_matmul_body(u0_ref, u1_ref, d0_ref, d1_ref, b_ref, w_ref, as_ref,
                      ad_ref, h_ref, esed_ref):
  den = d0_ref[...] + d1_ref[...]
  den = jnp.where(den > 0, den, 1.0)
  xin = (u0_ref[...] + u1_ref[...]) / den + b_ref[0:1, :]
  xin = jnp.where(xin > 0, xin, 0.01 * xin)
  h = jnp.dot(xin, w_ref[...], preferred_element_type=f32)
  es = jnp.sum(h * as_ref[0:1, :], axis=1, keepdims=True)
  ed = jnp.sum(h * ad_ref[0:1, :], axis=1, keepdims=True)
  z = jnp.zeros((h.shape[0], 6), f32)
  esed_ref[...] = jnp.concatenate([es, ed, z], axis=1)
  h_ref[...] = h


def _tc_norm_matmul(u0, u1, d0, d1, b, w, a_s, a_d):
  bn = 1024
  g = NPAD // bn
  din, dout = w.shape
  return pl.pallas_call(
      _norm_matmul_body,
      grid=(g,),
      in_specs=[
          pl.BlockSpec((bn, din), lambda i: (i, 0)),
          pl.BlockSpec((bn, din), lambda i: (i, 0)),
          pl.BlockSpec((bn, 1), lambda i: (i, 0)),
          pl.BlockSpec((bn, 1), lambda i: (i, 0)),
          pl.BlockSpec((8, din), lambda i: (0, 0)),
          pl.BlockSpec((din, dout), lambda i: (0, 0)),
          pl.BlockSpec((8, dout), lambda i: (0, 0)),
          pl.BlockSpec((8, dout), lambda i: (0, 0)),
      ],
      out_specs=[
          pl.BlockSpec((bn, dout), lambda i: (i, 0)),
          pl.BlockSpec((bn, 8), lambda i: (i, 0)),
      ],
      out_shape=[
          jax.ShapeDtypeStruct((NPAD, dout), f32),
          jax.ShapeDtypeStruct((NPAD, 8), f32),
      ],
  )(u0, u1, d0, d1, b, w, a_s, a_d)


def _final_body(u0_ref, u1_ref, d0_ref, d1_ref, b_ref, out_ref):
  den = d0_ref[...] + d1_ref[...]
  den = jnp.where(den > 0, den, 1.0)
  out_ref[...] = (u0_ref[...] + u1_ref[...])[:, :40] / den + b_ref[0:1, :40]


def _tc_final(u0, u1, d0, d1, b):
  bn = 1000
  return pl.pallas_call(
      _final_body,
      grid=(10,),
      in_specs=[
          pl.BlockSpec((bn, 48), lambda i: (i, 0)),
          pl.BlockSpec((bn, 48), lambda i: (i, 0)),
          pl.BlockSpec((bn, 1), lambda i: (i, 0)),
          pl.BlockSpec((bn, 1), lambda i: (i, 0)),
          pl.BlockSpec((8, 48), lambda i: (0, 0)),
      ],
      out_specs=pl.BlockSpec((bn, 40), lambda i: (i, 0)),
      out_shape=jax.ShapeDtypeStruct((N, 40), f32),
  )(u0, u1, d0, d1, b)


def _sc_edge_body(refs, *, d2, ngrp, nch, normalize):
  if normalize:
    (h_hbm, srcm, dstm, esedt, b_hbm, x2_hbm,
     es_v, ed_v, sidx2, didx2, gidx, rows, exall, exb, b_v,
     u_sh, den_sh, *sems) = refs
  else:
    (h_hbm, srcm, dstm, esedt, u_hbm, den_hbm,
     es_v, ed_v, sidx2, didx2, gidx, rows, exall, exb,
     u_sh, den_sh, *sems) = refs
  semg = sems[0:NBUF]
  semsc = sems[NBUF:2 * NBUF]
  semd = sems[2 * NBUF:2 * NBUF + 2]
  c = lax.axis_index("c")
  s = lax.axis_index("s")
  nvec = d2 // 16
  feature_split = ngrp > 1

  chunk_base = s * nch if feature_split else (c * NS + s) * nch
  pltpu.sync_copy(srcm.at[pl.ds(chunk_base, nch)], sidx2)
  pltpu.sync_copy(dstm.at[pl.ds(chunk_base, nch)], didx2)
  pltpu.sync_copy(esedt.at[0], es_v)
  pltpu.sync_copy(esedt.at[1], ed_v)

  def _lanemax(v_ref):
    def mxi(i, cur):
      return jnp.maximum(cur, v_ref[pl.ds(i * 16, 16)])
    return lax.fori_loop(1, NPAD // 16, mxi, v_ref[pl.ds(0, 16)])

  iota16 = lax.iota(jnp.int32, 16)
  def _bfly(m):
    for step in (8, 4, 2, 1):
      exb[pl.ds(0, 16)] = m
      m = jnp.maximum(m, plsc.load_gather(exb, [jnp.bitwise_xor(iota16, step)]))
    return m
  mbar = jnp.maximum(_bfly(_lanemax(es_v)) + _bfly(_lanemax(ed_v)),
                     jnp.zeros((16,), f32))

  zv = jnp.zeros((16,), f32)
  r0 = s * ROWS_PER_TILE

  @plsc.parallel_loop(0, nch, 1, unroll=2)
  def _(j):
    for i in range(CH // 16):
      sl = pl.ds(i * 16, 16)
      e = (plsc.load_gather(es_v, [sidx2[j, sl]]) +
           plsc.load_gather(ed_v, [didx2[j, sl]]))
      e = jnp.where(e > 0, e, 0.2 * e)
      exall[j, sl] = jnp.exp(e - mbar)

  def _issue_gather(jj, b, goff):
    @plsc.parallel_loop(0, CH // 16, 1, unroll=4)
    def _(i):
      sl = pl.ds(i * 16, 16)
      gidx[b, sl] = sidx2[jj, sl] + goff
    pltpu.async_copy(h_hbm.at[gidx.at[b]], rows.at[b], semg[b])

  for g in range(ngrp):
    grp = c * ngrp + g
    goff = grp * NPAD if feature_split else 0

    def _zrows(i, _):
      for bb in range(NBUF):
        for v in range(nvec):
          rows[bb, i, pl.ds(v * 16, 16)] = zv
      return 0
    lax.fori_loop(0, CH, _zrows, 0)
    def _zexb(i, _):
      exb[pl.ds(i * 16, 16)] = zv
      return 0
    lax.fori_loop(0, CH // 16, _zexb, 0)
    for bi in range(ROWS_PER_TILE // CH):
      pltpu.sync_copy(rows.at[0], u_sh.at[pl.ds(r0 + bi * CH, CH), :])
      if g == 0:
        pltpu.sync_copy(exb, den_sh.at[pl.ds(r0 + bi * CH, CH)])
    plsc.subcore_barrier()

    _issue_gather(0, 0, goff)
    _issue_gather(1, 1, goff)

    def quad_body(i, _):
      for t in range(NBUF):
        j = i * NBUF + t
        bn2 = (t + 2) % NBUF
        @pl.when(j >= 2)
        def _():
          pltpu.make_async_copy(rows.at[bn2], u_sh.at[didx2.at[j - 2]],
                                semsc[bn2]).wait()
        @pl.when(j + 2 < nch)
        def _():
          _issue_gather(j + 2, bn2, goff)

        pltpu.make_async_copy(h_hbm.at[gidx.at[t]], rows.at[t],
                              semg[t]).wait()
        bj = jnp.broadcast_to(j, (16,)).astype(jnp.int32)
        @plsc.parallel_loop(0, CH, 1, unroll=8)
        def _(k):
          bk = jnp.broadcast_to(k, (16,)).astype(jnp.int32)
          sv = plsc.load_gather(exall, [bj, bk])
          for v in range(nvec):
            sl = pl.ds(v * 16, 16)
            rows[t, k, sl] = rows[t, k, sl] * sv

        if g == 0:
          @pl.when(j >= 2)
          def _():
            pltpu.make_async_copy(exall.at[j - 2], den_sh.at[didx2.at[j - 2]],
                                  semd[t % 2]).wait()
        pltpu.async_copy(rows.at[t], u_sh.at[didx2.at[j]], semsc[t], add=True)
        if g == 0:
          pltpu.async_copy(exall.at[j], den_sh.at[didx2.at[j]], semd[t % 2],
                           add=True)
      return 0

    lax.fori_loop(0, nch // NBUF, quad_body, 0)
    for j in (nch - 2, nch - 1):
      bb = j % NBUF
      pltpu.make_async_copy(rows.at[bb], u_sh.at[didx2.at[j]],
                            semsc[bb]).wait()
      if g == 0:
        pltpu.make_async_copy(exall.at[j], den_sh.at[didx2.at[j]],
                              semd[j % 2]).wait()
    plsc.subcore_barrier()

    if normalize:
      pltpu.sync_copy(b_hbm.at[0, pl.ds(grp * d2, d2)], b_v)
      bvecs = [b_v[pl.ds(v * 16, 16)] for v in range(nvec)]
    for bi in range(ROWS_PER_TILE // CH):
      rsl = pl.ds(r0 + bi * CH, CH)
      pltpu.sync_copy(u_sh.at[rsl, :], rows.at[0])
      pltpu.sync_copy(den_sh.at[rsl], exb)
      if normalize:
        def rdi(i, _):
          sl = pl.ds(i * 16, 16)
          d = exb[sl]
          exb[sl] = 1.0 / jnp.where(d > 0, d, 1.0)
          return 0
        lax.fori_loop(0, CH // 16, rdi, 0)

        @plsc.parallel_loop(0, CH, 1, unroll=4)
        def _(k):
          bk = jnp.broadcast_to(k, (16,)).astype(jnp.int32)
          rv = plsc.load_gather(exb, [bk])
          for v in range(nvec):
            sl = pl.ds(v * 16, 16)
            t = rows[0, k, sl] * rv + bvecs[v]
            rows[0, k, sl] = jnp.where(t > 0, t, 0.01 * t)
        pltpu.sync_copy(rows.at[0], x2_hbm.at[grp, rsl, :])
      else:
        pltpu.sync_copy(rows.at[0], u_hbm.at[c, rsl, :])
        pltpu.sync_copy(exb, den_hbm.at[c, rsl])


def _sc_layer(h, srcm, dstm, esedt, b=None, *, d2, ngrp, normalize):
  feature_split = ngrp > 1
  nch = (EPAD // CH) // (NS if feature_split else (NS * NC))
  mesh = plsc.VectorSubcoreMesh(core_axis_name="c", subcore_axis_name="s",
                                num_cores=NC, num_subcores=NS)
  if normalize:
    out_type = jax.ShapeDtypeStruct((NC * ngrp, NPAD, d2), f32)
  else:
    out_type = (jax.ShapeDtypeStruct((2, NPAD, d2), f32),
                jax.ShapeDtypeStruct((2, NPAD), f32))
  scratch = [
      pltpu.VMEM((NPAD,), f32),
      pltpu.VMEM((NPAD,), f32),
      pltpu.VMEM((nch, CH), jnp.int32),
      pltpu.VMEM((nch, CH), jnp.int32),
      pltpu.VMEM((NBUF, CH), jnp.int32),
      pltpu.VMEM((NBUF, CH, d2), f32),
      pltpu.VMEM((nch, CH), f32),
      pltpu.VMEM((CH,), f32),
  ]
  if normalize:
    scratch.append(pltpu.VMEM((d2,), f32))
  scratch += [
      pltpu.VMEM_SHARED((NPAD, d2), f32),
      pltpu.VMEM_SHARED((NPAD,), f32),
  ] + [pltpu.SemaphoreType.DMA] * (2 * NBUF + 2)

  def body(*refs):
    _sc_edge_body(refs, d2=d2, ngrp=ngrp, nch=nch, normalize=normalize)

  cp = pltpu.CompilerParams(
      needs_layout_passes=False,
      use_tc_tiling_on_sc=False)
  fn = pl.kernel(body, out_type=out_type, mesh=mesh, scratch_types=scratch,
                 compiler_params=cp)
  if normalize:
    return fn(h, srcm, dstm, esedt, b)
  return fn(h, srcm, dstm, esedt)


def kernel(x, edge_index, W1, a_src1, a_dst1, b1, W2, a_src2, a_dst2, b2,
           W3, a_src3, a_dst3, b3):
  src = edge_index[0]
  dst = edge_index[1]
  npad_e = EPAD - E
  padidx = (jnp.arange(npad_e, dtype=jnp.int32) % (NPAD - N)) + N
  srcm = jnp.concatenate([src, padidx]).reshape(EPAD // CH, CH)
  dstm = jnp.concatenate([dst, padidx]).reshape(EPAD // CH, CH)
  xp = jnp.pad(x, ((0, NPAD - N), (0, 0)))

  bc8 = lambda v: jnp.broadcast_to(v[None, :], (8, v.shape[0]))
  w3p = jnp.pad(W3, ((0, 0), (0, 8)))
  a_src3p = jnp.pad(a_src3, (0, 8))
  a_dst3p = jnp.pad(a_dst3, (0, 8))
  b3p = jnp.pad(b3, (0, 8))

  h1, esed1 = _tc_matmul(xp, W1, bc8(a_src1), bc8(a_dst1), split=True)
  x2_8d = _sc_layer(h1.reshape(8 * NPAD, 32), srcm, dstm, esed1.T,
                    bc8(b1), d2=32, ngrp=4, normalize=True)

  h2, esed2 = _tc_matmul_split_in(x2_8d, W2, bc8(a_src2), bc8(a_dst2))
  u2, den2 = _sc_layer(h2, srcm, dstm, esed2.T,
                       d2=32, ngrp=1, normalize=False)

  h3, esed3 = _tc_norm_matmul(u2[0], u2[1], den2[0][:, None], den2[1][:, None],
                              bc8(b2), w3p, bc8(a_src3p), bc8(a_dst3p))
  u3, den3 = _sc_layer(h3, srcm, dstm, esed3.T,
                       d2=48, ngrp=1, normalize=False)

  return _tc_final(u3[0], u3[1], den3[0][:, None], den3[1][:, None], bc8(b3p))


# speedup vs baseline: 68.1952x; 1.3786x over previous
import functools

import jax
import jax.numpy as jnp
from jax import lax
from jax.experimental import pallas as pl
from jax.experimental.pallas import tpu as pltpu
from jax.experimental.pallas import tpu_sc as plsc

N = 10000
NPAD = 10240
E = 320000
EPAD = 327680
NC, NS = 2, 16
ROWS_PER_TILE = NPAD // NS
CH = 128
NBUF = 4

f32 = jnp.float32


def _esed(h, as_ref, ad_ref):
  es = jnp.sum(h * as_ref[0:1, :], axis=1, keepdims=True)
  ed = jnp.sum(h * ad_ref[0:1, :], axis=1, keepdims=True)
  z = jnp.zeros((h.shape[0], 6), f32)
  return jnp.concatenate([es, ed, z], axis=1)


def _esed1_body(x_ref, w_ref, as_ref, ad_ref, esed_ref):
  h = jnp.dot(x_ref[...], w_ref[...], preferred_element_type=f32)
  esed_ref[...] = _esed(h, as_ref, ad_ref)


def _tc_esed1(x, w, a_s, a_d):
  bn = 1024
  din, dout = w.shape
  return pl.pallas_call(
      _esed1_body,
      grid=(NPAD // bn,),
      in_specs=[
          pl.BlockSpec((bn, din), lambda i: (i, 0)),
          pl.BlockSpec((din, dout), lambda i: (0, 0)),
          pl.BlockSpec((8, dout), lambda i: (0, 0)),
          pl.BlockSpec((8, dout), lambda i: (0, 0)),
      ],
      out_specs=pl.BlockSpec((bn, 8), lambda i: (i, 0)),
      out_shape=jax.ShapeDtypeStruct((NPAD, 8), f32),
  )(x, w, a_s, a_d)


def _l2_body(v_ref, w1_ref, b1_ref, w2_ref, as_ref, ad_ref, h2_ref, esed_ref,
             *, ng):
  acc = jnp.dot(v_ref[0], w1_ref[0], preferred_element_type=f32)
  for q in range(1, ng):
    acc += jnp.dot(v_ref[q], w1_ref[q], preferred_element_type=f32)
  x2 = acc + b1_ref[0:1, :]
  x2 = jnp.where(x2 > 0, x2, 0.01 * x2)
  h2 = jnp.dot(x2, w2_ref[...], preferred_element_type=f32)
  esed_ref[...] = _esed(h2, as_ref, ad_ref)
  h2_ref[...] = h2


def _tc_l2(v, w1, b1, w2, a_s, a_d):
  bn = 1024
  ng = v.shape[0]
  dq = 128 // ng
  d2 = w2.shape[1]
  w1r = w1.reshape(ng, dq, w1.shape[1])
  return pl.pallas_call(
      functools.partial(_l2_body, ng=ng),
      grid=(NPAD // bn,),
      in_specs=[
          pl.BlockSpec((ng, bn, dq), lambda i: (0, i, 0)),
          pl.BlockSpec((ng, dq, 256), lambda i: (0, 0, 0)),
          pl.BlockSpec((8, 256), lambda i: (0, 0)),
          pl.BlockSpec((256, d2), lambda i: (0, 0)),
          pl.BlockSpec((8, d2), lambda i: (0, 0)),
          pl.BlockSpec((8, d2), lambda i: (0, 0)),
      ],
      out_specs=[
          pl.BlockSpec((bn, d2), lambda i: (i, 0)),
          pl.BlockSpec((bn, 8), lambda i: (i, 0)),
      ],
      out_shape=[
          jax.ShapeDtypeStruct((NPAD, d2), f32),
          jax.ShapeDtypeStruct((NPAD, 8), f32),
      ],
  )(v, w1r, b1, w2, a_s, a_d)


def _l3_body(u0_ref, u1_ref, d0_ref, d1_ref, b2_ref, w3_ref, as_ref, ad_ref,
             x3_ref, esed_ref):
  den = d0_ref[...] + d1_ref[...]
  den = jnp.where(den > 0, den, 1.0)
  x3 = (u0_ref[...] + u1_ref[...]) / den + b2_ref[0:1, :]
  x3 = jnp.where(x3 > 0, x3, 0.01 * x3)
  h3 = jnp.dot(x3, w3_ref[...], preferred_element_type=f32)
  esed_ref[...] = _esed(h3, as_ref, ad_ref)
  x3_ref[...] = x3


def _tc_l3(u0, u1, d0, d1, b2, w3, a_s, a_d):
  bn = 1024
  return pl.pallas_call(
      _l3_body,
      grid=(NPAD // bn,),
      in_specs=[
          pl.BlockSpec((bn, 32), lambda i: (i, 0)),
          pl.BlockSpec((bn, 32), lambda i: (i, 0)),
          pl.BlockSpec((bn, 1), lambda i: (i, 0)),
          pl.BlockSpec((bn, 1), lambda i: (i, 0)),
          pl.BlockSpec((8, 32), lambda i: (0, 0)),
          pl.BlockSpec((32, 48), lambda i: (0, 0)),
          pl.BlockSpec((8, 48), lambda i: (0, 0)),
          pl.BlockSpec((8, 48), lambda i: (0, 0)),
      ],
      out_specs=[
          pl.BlockSpec((bn, 32), lambda i: (i, 0)),
          pl.BlockSpec((bn, 8), lambda i: (i, 0)),
      ],
      out_shape=[
          jax.ShapeDtypeStruct((NPAD, 32), f32),
          jax.ShapeDtypeStruct((NPAD, 8), f32),
      ],
  )(u0, u1, d0, d1, b2, w3, a_s, a_d)


def _final_body(v0_ref, v1_ref, d0_ref, d1_ref, b3_ref, w3_ref, out_ref):
  den = d0_ref[...] + d1_ref[...]
  den = jnp.where(den > 0, den, 1.0)
  v = (v0_ref[...] + v1_ref[...]) / den
  h = jnp.dot(v, w3_ref[...], preferred_element_type=f32)
  out_ref[...] = h[:, :40] + b3_ref[0:1, :40]


def _tc_final(v0, v1, d0, d1, b3, w3):
  bn = 1000
  return pl.pallas_call(
      _final_body,
      grid=(10,),
      in_specs=[
          pl.BlockSpec((bn, 32), lambda i: (i, 0)),
          pl.BlockSpec((bn, 32), lambda i: (i, 0)),
          pl.BlockSpec((bn, 1), lambda i: (i, 0)),
          pl.BlockSpec((bn, 1), lambda i: (i, 0)),
          pl.BlockSpec((8, 48), lambda i: (0, 0)),
          pl.BlockSpec((32, 48), lambda i: (0, 0)),
      ],
      out_specs=pl.BlockSpec((bn, 40), lambda i: (i, 0)),
      out_shape=jax.ShapeDtypeStruct((N, 40), f32),
  )(v0, v1, d0, d1, b3, w3)


def _sc_edge_body(refs, *, d2, ngrp, fsplit, div):
  nch = (EPAD // CH) // (NS if fsplit else (NS * NC))
  if div:
    (h_hbm, srcm, dstm, esedt, x2_hbm,
     es_v, ed_v, sidx2, didx2, gidx, rows, exall, exb,
     u_sh, den_sh, *sems) = refs
  else:
    (h_hbm, srcm, dstm, esedt, u_hbm, den_hbm,
     es_v, ed_v, sidx2, didx2, gidx, rows, exall, exb,
     u_sh, den_sh, *sems) = refs
  semg = sems[0:NBUF]
  semsc = sems[NBUF:2 * NBUF]
  semd = sems[2 * NBUF:2 * NBUF + 2]
  c = lax.axis_index("c")
  s = lax.axis_index("s")
  nvec = d2 // 16

  chunk_base = s * nch if fsplit else (c * NS + s) * nch
  pltpu.sync_copy(srcm.at[pl.ds(chunk_base, nch)], sidx2)
  pltpu.sync_copy(dstm.at[pl.ds(chunk_base, nch)], didx2)
  pltpu.sync_copy(esedt.at[0], es_v)
  pltpu.sync_copy(esedt.at[1], ed_v)

  def _lanemax(v_ref):
    def mxi(i, cur):
      return jnp.maximum(cur, v_ref[pl.ds(i * 16, 16)])
    return lax.fori_loop(1, NPAD // 16, mxi, v_ref[pl.ds(0, 16)])

  iota16 = lax.iota(jnp.int32, 16)
  def _bfly(m):
    for step in (8, 4, 2, 1):
      exb[pl.ds(0, 16)] = m
      m = jnp.maximum(m, plsc.load_gather(exb, [jnp.bitwise_xor(iota16, step)]))
    return m
  mbar = jnp.maximum(_bfly(_lanemax(es_v)) + _bfly(_lanemax(ed_v)),
                     jnp.zeros((16,), f32))

  zv = jnp.zeros((16,), f32)
  r0 = s * ROWS_PER_TILE

  @plsc.parallel_loop(0, nch, 1, unroll=2)
  def _(j):
    for i in range(CH // 16):
      sl = pl.ds(i * 16, 16)
      e = (plsc.load_gather(es_v, [sidx2[j, sl]]) +
           plsc.load_gather(ed_v, [didx2[j, sl]]))
      e = jnp.where(e > 0, e, 0.2 * e)
      exall[j, sl] = jnp.exp(e - mbar)

  def _issue_gather(jj, b, goff):
    @plsc.parallel_loop(0, CH // 16, 1, unroll=4)
    def _(i):
      sl = pl.ds(i * 16, 16)
      gidx[b, sl] = sidx2[jj, sl] + goff
    pltpu.async_copy(h_hbm.at[gidx.at[b]], rows.at[b], semg[b])

  for g in range(ngrp):
    grp = c * ngrp + g
    goff = grp * NPAD if fsplit else 0

    def _zrows(i, _):
      for bb in range(NBUF):
        for v in range(nvec):
          rows[bb, i, pl.ds(v * 16, 16)] = zv
      return 0
    lax.fori_loop(0, CH, _zrows, 0)
    def _zexb(i, _):
      exb[pl.ds(i * 16, 16)] = zv
      return 0
    lax.fori_loop(0, CH // 16, _zexb, 0)
    for bi in range(ROWS_PER_TILE // CH):
      pltpu.sync_copy(rows.at[0], u_sh.at[pl.ds(r0 + bi * CH, CH), :])
      if g == 0:
        pltpu.sync_copy(exb, den_sh.at[pl.ds(r0 + bi * CH, CH)])
    plsc.subcore_barrier()

    _issue_gather(0, 0, goff)
    _issue_gather(1, 1, goff)

    def quad_body(i, _):
      for t in range(NBUF):
        j = i * NBUF + t
        bn2 = (t + 2) % NBUF
        @pl.when(j >= 2)
        def _():
          pltpu.make_async_copy(rows.at[bn2], u_sh.at[didx2.at[j - 2]],
                                semsc[bn2]).wait()
        @pl.when(j + 2 < nch)
        def _():
          _issue_gather(j + 2, bn2, goff)

        pltpu.make_async_copy(h_hbm.at[gidx.at[t]], rows.at[t],
                              semg[t]).wait()
        bj = jnp.broadcast_to(j, (16,)).astype(jnp.int32)
        @plsc.parallel_loop(0, CH, 1, unroll=8)
        def _(k):
          bk = jnp.broadcast_to(k, (16,)).astype(jnp.int32)
          sv = plsc.load_gather(exall, [bj, bk])
          for v in range(nvec):
            sl = pl.ds(v * 16, 16)
            rows[t, k, sl] = rows[t, k, sl] * sv

        if g == 0:
          @pl.when(j >= 2)
          def _():
            pltpu.make_async_copy(exall.at[j - 2], den_sh.at[didx2.at[j - 2]],
                                  semd[t % 2]).wait()
        pltpu.async_copy(rows.at[t], u_sh.at[didx2.at[j]], semsc[t], add=True)
        if g == 0:
          pltpu.async_copy(exall.at[j], den_sh.at[didx2.at[j]], semd[t % 2],
                           add=True)
      return 0

    lax.fori_loop(0, nch // NBUF, quad_body, 0)
    for j in (nch - 2, nch - 1):
      bb = j % NBUF
      pltpu.make_async_copy(rows.at[bb], u_sh.at[didx2.at[j]],
                            semsc[bb]).wait()
      if g == 0:
        pltpu.make_async_copy(exall.at[j], den_sh.at[didx2.at[j]],
                              semd[j % 2]).wait()
    plsc.subcore_barrier()

    for bi in range(ROWS_PER_TILE // CH):
      rsl = pl.ds(r0 + bi * CH, CH)
      pltpu.sync_copy(u_sh.at[rsl, :], rows.at[0])
      if div:
        pltpu.sync_copy(den_sh.at[rsl], exb)
        def rdi(i, _):
          sl = pl.ds(i * 16, 16)
          d = exb[sl]
          exb[sl] = 1.0 / jnp.where(d > 0, d, 1.0)
          return 0
        lax.fori_loop(0, CH // 16, rdi, 0)

        @plsc.parallel_loop(0, CH, 1, unroll=4)
        def _(k):
          bk = jnp.broadcast_to(k, (16,)).astype(jnp.int32)
          rv = plsc.load_gather(exb, [bk])
          for v in range(nvec):
            sl = pl.ds(v * 16, 16)
            rows[0, k, sl] = rows[0, k, sl] * rv
        pltpu.sync_copy(rows.at[0], x2_hbm.at[grp, rsl, :])
      else:
        pltpu.sync_copy(rows.at[0], u_hbm.at[c, rsl, :])
        pltpu.sync_copy(den_sh.at[rsl], exb)
        pltpu.sync_copy(exb, den_hbm.at[c, rsl])


def _sc_layer(h, srcm, dstm, esedt, *, d2, ngrp, fsplit, div):
  nch = (EPAD // CH) // (NS if fsplit else (NS * NC))
  mesh = plsc.VectorSubcoreMesh(core_axis_name="c", subcore_axis_name="s",
                                num_cores=NC, num_subcores=NS)
  if div:
    out_type = jax.ShapeDtypeStruct((NC * ngrp, NPAD, d2), f32)
  else:
    out_type = (jax.ShapeDtypeStruct((2, NPAD, d2), f32),
                jax.ShapeDtypeStruct((2, NPAD), f32))
  scratch = [
      pltpu.VMEM((NPAD,), f32),
      pltpu.VMEM((NPAD,), f32),
      pltpu.VMEM((nch, CH), jnp.int32),
      pltpu.VMEM((nch, CH), jnp.int32),
      pltpu.VMEM((NBUF, CH), jnp.int32),
      pltpu.VMEM((NBUF, CH, d2), f32),
      pltpu.VMEM((nch, CH), f32),
      pltpu.VMEM((CH,), f32),
      pltpu.VMEM_SHARED((NPAD, d2), f32),
      pltpu.VMEM_SHARED((NPAD,), f32),
  ] + [pltpu.SemaphoreType.DMA] * (2 * NBUF + 2)

  def body(*refs):
    _sc_edge_body(refs, d2=d2, ngrp=ngrp, fsplit=fsplit, div=div)

  cp = pltpu.CompilerParams(
      needs_layout_passes=False,
      use_tc_tiling_on_sc=False)
  fn = pl.kernel(body, out_type=out_type, mesh=mesh, scratch_types=scratch,
                 compiler_params=cp)
  return fn(h, srcm, dstm, esedt)


def kernel(x, edge_index, W1, a_src1, a_dst1, b1, W2, a_src2, a_dst2, b2,
           W3, a_src3, a_dst3, b3):
  src = edge_index[0]
  dst = edge_index[1]
  npad_e = EPAD - E
  padidx = (jnp.arange(npad_e, dtype=jnp.int32) % (NPAD - N)) + N
  srcm = jnp.concatenate([src, padidx]).reshape(EPAD // CH, CH)
  dstm = jnp.concatenate([dst, padidx]).reshape(EPAD // CH, CH)
  xp = jnp.pad(x, ((0, NPAD - N), (0, 0)))

  bc8 = lambda v: jnp.broadcast_to(v[None, :], (8, v.shape[0]))
  w3p = jnp.pad(W3, ((0, 0), (0, 8)))
  a_src3p = jnp.pad(a_src3, (0, 8))
  a_dst3p = jnp.pad(a_dst3, (0, 8))
  b3p = jnp.pad(b3, (0, 8))

  L1G = 2
  ngroups = NC * L1G
  dq = 128 // ngroups
  esed1 = _tc_esed1(xp, W1, bc8(a_src1), bc8(a_dst1))
  x1g = xp.reshape(NPAD, ngroups, dq).transpose(1, 0, 2)
  v1 = _sc_layer(x1g.reshape(ngroups * NPAD, dq), srcm, dstm, esed1.T,
                 d2=dq, ngrp=L1G, fsplit=True, div=True)

  h2, esed2 = _tc_l2(v1, W1, bc8(b1), W2, bc8(a_src2), bc8(a_dst2))
  u2, den2 = _sc_layer(h2, srcm, dstm, esed2.T,
                       d2=32, ngrp=1, fsplit=False, div=False)

  x3, esed3 = _tc_l3(u2[0], u2[1], den2[0][:, None], den2[1][:, None],
                     bc8(b2), w3p, bc8(a_src3p), bc8(a_dst3p))
  v3, den3 = _sc_layer(x3, srcm, dstm, esed3.T,
                       d2=32, ngrp=1, fsplit=False, div=False)

  return _tc_final(v3[0], v3[1], den3[0][:, None], den3[1][:, None],
                   bc8(b3p), w3p)


# speedup vs baseline: 69.6339x; 1.0211x over previous
import functools

import jax
import jax.numpy as jnp
from jax import lax
from jax.experimental import pallas as pl
from jax.experimental.pallas import tpu as pltpu
from jax.experimental.pallas import tpu_sc as plsc

N = 10000
NPAD = 10240
E = 320000
EPAD = 327680
NC, NS = 2, 16
ROWS_PER_TILE = NPAD // NS
CH = 128
NBUF = 4

f32 = jnp.float32


def _esed(h, as_ref, ad_ref):
  es = jnp.sum(h * as_ref[0:1, :], axis=1, keepdims=True)
  ed = jnp.sum(h * ad_ref[0:1, :], axis=1, keepdims=True)
  z = jnp.zeros((h.shape[0], 6), f32)
  return jnp.concatenate([es, ed, z], axis=1)


def _esed1_body(x_ref, w_ref, as_ref, ad_ref, esed_ref, xg_ref, *, ngroups):
  h = jnp.dot(x_ref[...], w_ref[...], preferred_element_type=f32)
  esed_ref[...] = _esed(h, as_ref, ad_ref)
  dq = 128 // ngroups
  for q in range(ngroups):
    xg_ref[q] = x_ref[:, q * dq:(q + 1) * dq]


def _tc_esed1(x, w, a_s, a_d, ngroups):
  bn = 1024
  din, dout = w.shape
  dq = din // ngroups
  return pl.pallas_call(
      functools.partial(_esed1_body, ngroups=ngroups),
      grid=(NPAD // bn,),
      in_specs=[
          pl.BlockSpec((bn, din), lambda i: (i, 0)),
          pl.BlockSpec((din, dout), lambda i: (0, 0)),
          pl.BlockSpec((8, dout), lambda i: (0, 0)),
          pl.BlockSpec((8, dout), lambda i: (0, 0)),
      ],
      out_specs=[
          pl.BlockSpec((bn, 8), lambda i: (i, 0)),
          pl.BlockSpec((ngroups, bn, dq), lambda i: (0, i, 0)),
      ],
      out_shape=[
          jax.ShapeDtypeStruct((NPAD, 8), f32),
          jax.ShapeDtypeStruct((ngroups, NPAD, dq), f32),
      ],
  )(x, w, a_s, a_d)


def _l2_body(v_ref, w1_ref, b1_ref, w2_ref, as_ref, ad_ref, h2_ref, esed_ref,
             *, ng):
  acc = jnp.dot(v_ref[0], w1_ref[0], preferred_element_type=f32)
  for q in range(1, ng):
    acc += jnp.dot(v_ref[q], w1_ref[q], preferred_element_type=f32)
  x2 = acc + b1_ref[0:1, :]
  x2 = jnp.where(x2 > 0, x2, 0.01 * x2)
  h2 = jnp.dot(x2, w2_ref[...], preferred_element_type=f32)
  esed_ref[...] = _esed(h2, as_ref, ad_ref)
  h2_ref[...] = h2


def _tc_l2(v, w1, b1, w2, a_s, a_d):
  bn = 1024
  ng = v.shape[0]
  dq = 128 // ng
  d2 = w2.shape[1]
  w1r = w1.reshape(ng, dq, w1.shape[1])
  return pl.pallas_call(
      functools.partial(_l2_body, ng=ng),
      grid=(NPAD // bn,),
      in_specs=[
          pl.BlockSpec((ng, bn, dq), lambda i: (0, i, 0)),
          pl.BlockSpec((ng, dq, 256), lambda i: (0, 0, 0)),
          pl.BlockSpec((8, 256), lambda i: (0, 0)),
          pl.BlockSpec((256, d2), lambda i: (0, 0)),
          pl.BlockSpec((8, d2), lambda i: (0, 0)),
          pl.BlockSpec((8, d2), lambda i: (0, 0)),
      ],
      out_specs=[
          pl.BlockSpec((bn, d2), lambda i: (i, 0)),
          pl.BlockSpec((bn, 8), lambda i: (i, 0)),
      ],
      out_shape=[
          jax.ShapeDtypeStruct((NPAD, d2), f32),
          jax.ShapeDtypeStruct((NPAD, 8), f32),
      ],
  )(v, w1r, b1, w2, a_s, a_d)


def _l3_body(u0_ref, u1_ref, d0_ref, d1_ref, b2_ref, w3_ref, as_ref, ad_ref,
             x3_ref, esed_ref):
  den = d0_ref[...] + d1_ref[...]
  den = jnp.where(den > 0, den, 1.0)
  x3 = (u0_ref[...] + u1_ref[...]) / den + b2_ref[0:1, :]
  x3 = jnp.where(x3 > 0, x3, 0.01 * x3)
  h3 = jnp.dot(x3, w3_ref[...], preferred_element_type=f32)
  esed_ref[...] = _esed(h3, as_ref, ad_ref)
  x3_ref[...] = x3


def _tc_l3(u0, u1, d0, d1, b2, w3, a_s, a_d):
  bn = 1024
  return pl.pallas_call(
      _l3_body,
      grid=(NPAD // bn,),
      in_specs=[
          pl.BlockSpec((bn, 32), lambda i: (i, 0)),
          pl.BlockSpec((bn, 32), lambda i: (i, 0)),
          pl.BlockSpec((bn, 1), lambda i: (i, 0)),
          pl.BlockSpec((bn, 1), lambda i: (i, 0)),
          pl.BlockSpec((8, 32), lambda i: (0, 0)),
          pl.BlockSpec((32, 48), lambda i: (0, 0)),
          pl.BlockSpec((8, 48), lambda i: (0, 0)),
          pl.BlockSpec((8, 48), lambda i: (0, 0)),
      ],
      out_specs=[
          pl.BlockSpec((bn, 32), lambda i: (i, 0)),
          pl.BlockSpec((bn, 8), lambda i: (i, 0)),
      ],
      out_shape=[
          jax.ShapeDtypeStruct((NPAD, 32), f32),
          jax.ShapeDtypeStruct((NPAD, 8), f32),
      ],
  )(u0, u1, d0, d1, b2, w3, a_s, a_d)


def _final_body(v0_ref, v1_ref, d0_ref, d1_ref, b3_ref, w3_ref, out_ref):
  den = d0_ref[...] + d1_ref[...]
  den = jnp.where(den > 0, den, 1.0)
  v = (v0_ref[...] + v1_ref[...]) / den
  h = jnp.dot(v, w3_ref[...], preferred_element_type=f32)
  out_ref[...] = h[:, :40] + b3_ref[0:1, :40]


def _tc_final(v0, v1, d0, d1, b3, w3):
  bn = 1000
  return pl.pallas_call(
      _final_body,
      grid=(10,),
      in_specs=[
          pl.BlockSpec((bn, 32), lambda i: (i, 0)),
          pl.BlockSpec((bn, 32), lambda i: (i, 0)),
          pl.BlockSpec((bn, 1), lambda i: (i, 0)),
          pl.BlockSpec((bn, 1), lambda i: (i, 0)),
          pl.BlockSpec((8, 48), lambda i: (0, 0)),
          pl.BlockSpec((32, 48), lambda i: (0, 0)),
      ],
      out_specs=pl.BlockSpec((bn, 40), lambda i: (i, 0)),
      out_shape=jax.ShapeDtypeStruct((N, 40), f32),
  )(v0, v1, d0, d1, b3, w3)


def _sc_edge_body(refs, *, d2, ngrp, fsplit, div):
  nch = (EPAD // CH) // (NS if fsplit else (NS * NC))
  if div:
    (h_hbm, srcm, dstm, esedt, x2_hbm,
     es_v, ed_v, sidx2, didx2, gidx, rows, exall, exb,
     u_sh, den_sh, *sems) = refs
  else:
    (h_hbm, srcm, dstm, esedt, u_hbm, den_hbm,
     es_v, ed_v, sidx2, didx2, gidx, rows, exall, exb,
     u_sh, den_sh, *sems) = refs
  semg = sems[0:NBUF]
  semsc = sems[NBUF:2 * NBUF]
  semd = sems[2 * NBUF:2 * NBUF + 2]
  c = lax.axis_index("c")
  s = lax.axis_index("s")
  nvec = d2 // 16

  chunk_base = s * nch if fsplit else (c * NS + s) * nch
  pltpu.sync_copy(srcm.at[pl.ds(chunk_base, nch)], sidx2)
  pltpu.sync_copy(dstm.at[pl.ds(chunk_base, nch)], didx2)
  pltpu.sync_copy(esedt.at[0], es_v)
  pltpu.sync_copy(esedt.at[1], ed_v)

  def _lanemax(v_ref):
    def mxi(i, cur):
      return jnp.maximum(cur, v_ref[pl.ds(i * 16, 16)])
    return lax.fori_loop(1, NPAD // 16, mxi, v_ref[pl.ds(0, 16)])

  iota16 = lax.iota(jnp.int32, 16)
  def _bfly(m):
    for step in (8, 4, 2, 1):
      exb[pl.ds(0, 16)] = m
      m = jnp.maximum(m, plsc.load_gather(exb, [jnp.bitwise_xor(iota16, step)]))
    return m
  mbar = jnp.maximum(_bfly(_lanemax(es_v)) + _bfly(_lanemax(ed_v)),
                     jnp.zeros((16,), f32))

  zv = jnp.zeros((16,), f32)
  r0 = s * ROWS_PER_TILE

  @plsc.parallel_loop(0, nch, 1, unroll=2)
  def _(j):
    for i in range(CH // 16):
      sl = pl.ds(i * 16, 16)
      e = (plsc.load_gather(es_v, [sidx2[j, sl]]) +
           plsc.load_gather(ed_v, [didx2[j, sl]]))
      e = jnp.where(e > 0, e, 0.2 * e)
      exall[j, sl] = jnp.exp(e - mbar)

  def _issue_gather(jj, b, goff):
    @plsc.parallel_loop(0, CH // 16, 1, unroll=4)
    def _(i):
      sl = pl.ds(i * 16, 16)
      gidx[b, sl] = sidx2[jj, sl] + goff
    pltpu.async_copy(h_hbm.at[gidx.at[b]], rows.at[b], semg[b])

  for g in range(ngrp):
    grp = c * ngrp + g
    goff = grp * NPAD if fsplit else 0

    def _zrows(i, _):
      for bb in range(NBUF):
        for v in range(nvec):
          rows[bb, i, pl.ds(v * 16, 16)] = zv
      return 0
    lax.fori_loop(0, CH, _zrows, 0)
    def _zexb(i, _):
      exb[pl.ds(i * 16, 16)] = zv
      return 0
    lax.fori_loop(0, CH // 16, _zexb, 0)
    for bi in range(ROWS_PER_TILE // CH):
      pltpu.sync_copy(rows.at[0], u_sh.at[pl.ds(r0 + bi * CH, CH), :])
      if g == 0:
        pltpu.sync_copy(exb, den_sh.at[pl.ds(r0 + bi * CH, CH)])
    plsc.subcore_barrier()

    _issue_gather(0, 0, goff)
    _issue_gather(1, 1, goff)

    def quad_body(i, _):
      for t in range(NBUF):
        j = i * NBUF + t
        bn2 = (t + 2) % NBUF
        @pl.when(j >= 2)
        def _():
          pltpu.make_async_copy(rows.at[bn2], u_sh.at[didx2.at[j - 2]],
                                semsc[bn2]).wait()
        @pl.when(j + 2 < nch)
        def _():
          _issue_gather(j + 2, bn2, goff)

        pltpu.make_async_copy(h_hbm.at[gidx.at[t]], rows.at[t],
                              semg[t]).wait()
        bj = jnp.broadcast_to(j, (16,)).astype(jnp.int32)
        @plsc.parallel_loop(0, CH, 1, unroll=8)
        def _(k):
          bk = jnp.broadcast_to(k, (16,)).astype(jnp.int32)
          sv = plsc.load_gather(exall, [bj, bk])
          for v in range(nvec):
            sl = pl.ds(v * 16, 16)
            rows[t, k, sl] = rows[t, k, sl] * sv

        if g == 0:
          @pl.when(j >= 2)
          def _():
            pltpu.make_async_copy(exall.at[j - 2], den_sh.at[didx2.at[j - 2]],
                                  semd[t % 2]).wait()
        pltpu.async_copy(rows.at[t], u_sh.at[didx2.at[j]], semsc[t], add=True)
        if g == 0:
          pltpu.async_copy(exall.at[j], den_sh.at[didx2.at[j]], semd[t % 2],
                           add=True)
      return 0

    lax.fori_loop(0, nch // NBUF, quad_body, 0)
    for j in (nch - 2, nch - 1):
      bb = j % NBUF
      pltpu.make_async_copy(rows.at[bb], u_sh.at[didx2.at[j]],
                            semsc[bb]).wait()
      if g == 0:
        pltpu.make_async_copy(exall.at[j], den_sh.at[didx2.at[j]],
                              semd[j % 2]).wait()
    plsc.subcore_barrier()

    for bi in range(ROWS_PER_TILE // CH):
      rsl = pl.ds(r0 + bi * CH, CH)
      pltpu.sync_copy(u_sh.at[rsl, :], rows.at[0])
      if div:
        pltpu.sync_copy(den_sh.at[rsl], exb)
        def rdi(i, _):
          sl = pl.ds(i * 16, 16)
          d = exb[sl]
          exb[sl] = 1.0 / jnp.where(d > 0, d, 1.0)
          return 0
        lax.fori_loop(0, CH // 16, rdi, 0)

        @plsc.parallel_loop(0, CH, 1, unroll=4)
        def _(k):
          bk = jnp.broadcast_to(k, (16,)).astype(jnp.int32)
          rv = plsc.load_gather(exb, [bk])
          for v in range(nvec):
            sl = pl.ds(v * 16, 16)
            rows[0, k, sl] = rows[0, k, sl] * rv
        pltpu.sync_copy(rows.at[0], x2_hbm.at[grp, rsl, :])
      else:
        pltpu.sync_copy(rows.at[0], u_hbm.at[c, rsl, :])
        pltpu.sync_copy(den_sh.at[rsl], exb)
        pltpu.sync_copy(exb, den_hbm.at[c, rsl])


def _sc_layer(h, srcm, dstm, esedt, *, d2, ngrp, fsplit, div):
  nch = (EPAD // CH) // (NS if fsplit else (NS * NC))
  mesh = plsc.VectorSubcoreMesh(core_axis_name="c", subcore_axis_name="s",
                                num_cores=NC, num_subcores=NS)
  if div:
    out_type = jax.ShapeDtypeStruct((NC * ngrp, NPAD, d2), f32)
  else:
    out_type = (jax.ShapeDtypeStruct((2, NPAD, d2), f32),
                jax.ShapeDtypeStruct((2, NPAD), f32))
  scratch = [
      pltpu.VMEM((NPAD,), f32),
      pltpu.VMEM((NPAD,), f32),
      pltpu.VMEM((nch, CH), jnp.int32),
      pltpu.VMEM((nch, CH), jnp.int32),
      pltpu.VMEM((NBUF, CH), jnp.int32),
      pltpu.VMEM((NBUF, CH, d2), f32),
      pltpu.VMEM((nch, CH), f32),
      pltpu.VMEM((CH,), f32),
      pltpu.VMEM_SHARED((NPAD, d2), f32),
      pltpu.VMEM_SHARED((NPAD,), f32),
  ] + [pltpu.SemaphoreType.DMA] * (2 * NBUF + 2)

  def body(*refs):
    _sc_edge_body(refs, d2=d2, ngrp=ngrp, fsplit=fsplit, div=div)

  cp = pltpu.CompilerParams(
      needs_layout_passes=False,
      use_tc_tiling_on_sc=False)
  fn = pl.kernel(body, out_type=out_type, mesh=mesh, scratch_types=scratch,
                 compiler_params=cp)
  return fn(h, srcm, dstm, esedt)


def kernel(x, edge_index, W1, a_src1, a_dst1, b1, W2, a_src2, a_dst2, b2,
           W3, a_src3, a_dst3, b3):
  src = edge_index[0]
  dst = edge_index[1]
  npad_e = EPAD - E
  padidx = (jnp.arange(npad_e, dtype=jnp.int32) % (NPAD - N)) + N
  srcm = jnp.concatenate([src, padidx]).reshape(EPAD // CH, CH)
  dstm = jnp.concatenate([dst, padidx]).reshape(EPAD // CH, CH)
  xp = jnp.pad(x, ((0, NPAD - N), (0, 0)))

  bc8 = lambda v: jnp.broadcast_to(v[None, :], (8, v.shape[0]))
  w3p = jnp.pad(W3, ((0, 0), (0, 8)))
  a_src3p = jnp.pad(a_src3, (0, 8))
  a_dst3p = jnp.pad(a_dst3, (0, 8))
  b3p = jnp.pad(b3, (0, 8))

  L1G = 2
  ngroups = NC * L1G
  dq = 128 // ngroups
  esed1, x1g = _tc_esed1(xp, W1, bc8(a_src1), bc8(a_dst1), ngroups)
  v1 = _sc_layer(x1g.reshape(ngroups * NPAD, dq), srcm, dstm, esed1.T,
                 d2=dq, ngrp=L1G, fsplit=True, div=True)

  h2, esed2 = _tc_l2(v1, W1, bc8(b1), W2, bc8(a_src2), bc8(a_dst2))
  u2, den2 = _sc_layer(h2, srcm, dstm, esed2.T,
                       d2=32, ngrp=1, fsplit=False, div=False)

  x3, esed3 = _tc_l3(u2[0], u2[1], den2[0][:, None], den2[1][:, None],
                     bc8(b2), w3p, bc8(a_src3p), bc8(a_dst3p))
  v3, den3 = _sc_layer(x3, srcm, dstm, esed3.T,
                       d2=32, ngrp=1, fsplit=False, div=False)

  return _tc_final(v3[0], v3[1], den3[0][:, None], den3[1][:, None],
                   bc8(b3p), w3p)


# speedup vs baseline: 71.5001x; 1.0268x over previous
import functools

import jax
import jax.numpy as jnp
from jax import lax
from jax.experimental import pallas as pl
from jax.experimental.pallas import tpu as pltpu
from jax.experimental.pallas import tpu_sc as plsc

N = 10000
NPAD = 10240
E = 320000
EPAD = 327680
NC, NS = 2, 16
ROWS_PER_TILE = NPAD // NS
CH = 128
NBUF = 4

f32 = jnp.float32


def _esed(h, as_ref, ad_ref):
  es = jnp.sum(h * as_ref[0:1, :], axis=1, keepdims=True)
  ed = jnp.sum(h * ad_ref[0:1, :], axis=1, keepdims=True)
  z = jnp.zeros((h.shape[0], 6), f32)
  return jnp.concatenate([es, ed, z], axis=1)


def _esed1_body(x_ref, w_ref, as_ref, ad_ref, esed_ref, xg_ref, *, ngroups):
  h = jnp.dot(x_ref[...], w_ref[...], preferred_element_type=f32)
  esed_ref[...] = _esed(h, as_ref, ad_ref)
  dq = 128 // ngroups
  for q in range(ngroups):
    xg_ref[q] = x_ref[:, q * dq:(q + 1) * dq]


def _tc_esed1(x, w, a_s, a_d, ngroups):
  bn = 1024
  din, dout = w.shape
  dq = din // ngroups
  return pl.pallas_call(
      functools.partial(_esed1_body, ngroups=ngroups),
      grid=(NPAD // bn,),
      in_specs=[
          pl.BlockSpec((bn, din), lambda i: (i, 0)),
          pl.BlockSpec((din, dout), lambda i: (0, 0)),
          pl.BlockSpec((8, dout), lambda i: (0, 0)),
          pl.BlockSpec((8, dout), lambda i: (0, 0)),
      ],
      out_specs=[
          pl.BlockSpec((bn, 8), lambda i: (i, 0)),
          pl.BlockSpec((ngroups, bn, dq), lambda i: (0, i, 0)),
      ],
      out_shape=[
          jax.ShapeDtypeStruct((NPAD, 8), f32),
          jax.ShapeDtypeStruct((ngroups, NPAD, dq), f32),
      ],
  )(x, w, a_s, a_d)


def _l2_body(v_ref, w1_ref, b1_ref, w2_ref, as_ref, ad_ref, h2_ref, esed_ref,
             *, ng):
  acc = jnp.dot(v_ref[0], w1_ref[0], preferred_element_type=f32)
  for q in range(1, ng):
    acc += jnp.dot(v_ref[q], w1_ref[q], preferred_element_type=f32)
  x2 = acc + b1_ref[0:1, :]
  x2 = jnp.where(x2 > 0, x2, 0.01 * x2)
  h2 = jnp.dot(x2, w2_ref[...], preferred_element_type=f32)
  esed_ref[...] = _esed(h2, as_ref, ad_ref)
  h2_ref[...] = h2


def _tc_l2(v, w1, b1, w2, a_s, a_d):
  bn = 1024
  ng = v.shape[0]
  dq = 128 // ng
  d2 = w2.shape[1]
  w1r = w1.reshape(ng, dq, w1.shape[1])
  return pl.pallas_call(
      functools.partial(_l2_body, ng=ng),
      grid=(NPAD // bn,),
      in_specs=[
          pl.BlockSpec((ng, bn, dq), lambda i: (0, i, 0)),
          pl.BlockSpec((ng, dq, 256), lambda i: (0, 0, 0)),
          pl.BlockSpec((8, 256), lambda i: (0, 0)),
          pl.BlockSpec((256, d2), lambda i: (0, 0)),
          pl.BlockSpec((8, d2), lambda i: (0, 0)),
          pl.BlockSpec((8, d2), lambda i: (0, 0)),
      ],
      out_specs=[
          pl.BlockSpec((bn, d2), lambda i: (i, 0)),
          pl.BlockSpec((bn, 8), lambda i: (i, 0)),
      ],
      out_shape=[
          jax.ShapeDtypeStruct((NPAD, d2), f32),
          jax.ShapeDtypeStruct((NPAD, 8), f32),
      ],
  )(v, w1r, b1, w2, a_s, a_d)


def _l3_body(u0_ref, u1_ref, d0_ref, d1_ref, b2_ref, w3_ref, as_ref, ad_ref,
             x3_ref, esed_ref):
  den = d0_ref[...] + d1_ref[...]
  den = jnp.where(den > 0, den, 1.0)
  x3 = (u0_ref[...] + u1_ref[...]) / den + b2_ref[0:1, :]
  x3 = jnp.where(x3 > 0, x3, 0.01 * x3)
  h3 = jnp.dot(x3, w3_ref[...], preferred_element_type=f32)
  esed_ref[...] = _esed(h3, as_ref, ad_ref)
  x3_ref[...] = x3


def _tc_l3(u0, u1, d0, d1, b2, w3, a_s, a_d):
  bn = 1024
  return pl.pallas_call(
      _l3_body,
      grid=(NPAD // bn,),
      in_specs=[
          pl.BlockSpec((bn, 32), lambda i: (i, 0)),
          pl.BlockSpec((bn, 32), lambda i: (i, 0)),
          pl.BlockSpec((bn, 1), lambda i: (i, 0)),
          pl.BlockSpec((bn, 1), lambda i: (i, 0)),
          pl.BlockSpec((8, 32), lambda i: (0, 0)),
          pl.BlockSpec((32, 48), lambda i: (0, 0)),
          pl.BlockSpec((8, 48), lambda i: (0, 0)),
          pl.BlockSpec((8, 48), lambda i: (0, 0)),
      ],
      out_specs=[
          pl.BlockSpec((bn, 32), lambda i: (i, 0)),
          pl.BlockSpec((bn, 8), lambda i: (i, 0)),
      ],
      out_shape=[
          jax.ShapeDtypeStruct((NPAD, 32), f32),
          jax.ShapeDtypeStruct((NPAD, 8), f32),
      ],
  )(u0, u1, d0, d1, b2, w3, a_s, a_d)


def _final_body(v0_ref, v1_ref, d0_ref, d1_ref, b3_ref, w3_ref, out_ref):
  den = d0_ref[...] + d1_ref[...]
  den = jnp.where(den > 0, den, 1.0)
  v = (v0_ref[...] + v1_ref[...]) / den
  h = jnp.dot(v, w3_ref[...], preferred_element_type=f32)
  out_ref[...] = h[:, :40] + b3_ref[0:1, :40]


def _tc_final(v0, v1, d0, d1, b3, w3):
  bn = 1000
  return pl.pallas_call(
      _final_body,
      grid=(10,),
      in_specs=[
          pl.BlockSpec((bn, 32), lambda i: (i, 0)),
          pl.BlockSpec((bn, 32), lambda i: (i, 0)),
          pl.BlockSpec((bn, 1), lambda i: (i, 0)),
          pl.BlockSpec((bn, 1), lambda i: (i, 0)),
          pl.BlockSpec((8, 48), lambda i: (0, 0)),
          pl.BlockSpec((32, 48), lambda i: (0, 0)),
      ],
      out_specs=pl.BlockSpec((bn, 40), lambda i: (i, 0)),
      out_shape=jax.ShapeDtypeStruct((N, 40), f32),
  )(v0, v1, d0, d1, b3, w3)


def _sc_edge_body(refs, *, d2, ngrp, fsplit, div):
  nch = (EPAD // CH) // (NS if fsplit else (NS * NC))
  if div:
    (h_hbm, srcm, dstm, esedt, x2_hbm,
     es_v, ed_v, sidx2, didx2, gidx, rows, exall, exb,
     u_sh, den_sh, *sems) = refs
  else:
    (h_hbm, srcm, dstm, esedt, u_hbm, den_hbm,
     es_v, ed_v, sidx2, didx2, gidx, rows, exall, exb,
     u_sh, den_sh, *sems) = refs
  semg = sems[0:NBUF]
  semsc = sems[NBUF:2 * NBUF]
  semd = sems[2 * NBUF:2 * NBUF + 2]
  c = lax.axis_index("c")
  s = lax.axis_index("s")
  nvec = d2 // 16

  chunk_base = s * nch if fsplit else (c * NS + s) * nch
  pltpu.sync_copy(srcm.at[pl.ds(chunk_base, nch)], sidx2)
  pltpu.sync_copy(dstm.at[pl.ds(chunk_base, nch)], didx2)
  pltpu.sync_copy(esedt.at[0], es_v)
  pltpu.sync_copy(esedt.at[1], ed_v)

  def _lanemax(v_ref):
    def mxi(i, cur):
      return tuple(jnp.maximum(cur[k], v_ref[pl.ds(i * 64 + k * 16, 16)])
                   for k in range(4))
    cs = lax.fori_loop(1, NPAD // 64, mxi,
                       tuple(v_ref[pl.ds(k * 16, 16)] for k in range(4)))
    return jnp.maximum(jnp.maximum(cs[0], cs[1]), jnp.maximum(cs[2], cs[3]))

  iota16 = lax.iota(jnp.int32, 16)
  def _bfly(m):
    for step in (8, 4, 2, 1):
      exb[pl.ds(0, 16)] = m
      m = jnp.maximum(m, plsc.load_gather(exb, [jnp.bitwise_xor(iota16, step)]))
    return m
  mbar = jnp.maximum(_bfly(_lanemax(es_v)) + _bfly(_lanemax(ed_v)),
                     jnp.zeros((16,), f32))

  zv = jnp.zeros((16,), f32)
  r0 = s * ROWS_PER_TILE

  @plsc.parallel_loop(0, nch, 1, unroll=2)
  def _(j):
    for i in range(CH // 16):
      sl = pl.ds(i * 16, 16)
      e = (plsc.load_gather(es_v, [sidx2[j, sl]]) +
           plsc.load_gather(ed_v, [didx2[j, sl]]))
      e = jnp.where(e > 0, e, 0.2 * e)
      exall[j, sl] = jnp.exp(e - mbar)

  def _issue_gather(jj, b, goff):
    @plsc.parallel_loop(0, CH // 16, 1, unroll=4)
    def _(i):
      sl = pl.ds(i * 16, 16)
      gidx[b, sl] = sidx2[jj, sl] + goff
    pltpu.async_copy(h_hbm.at[gidx.at[b]], rows.at[b], semg[b])

  for g in range(ngrp):
    grp = c * ngrp + g
    goff = grp * NPAD if fsplit else 0

    def _zrows(i, _):
      for bb in range(NBUF):
        for v in range(nvec):
          rows[bb, i, pl.ds(v * 16, 16)] = zv
      return 0
    lax.fori_loop(0, CH, _zrows, 0)
    def _zexb(i, _):
      exb[pl.ds(i * 16, 16)] = zv
      return 0
    lax.fori_loop(0, CH // 16, _zexb, 0)
    for bi in range(ROWS_PER_TILE // CH):
      pltpu.sync_copy(rows.at[0], u_sh.at[pl.ds(r0 + bi * CH, CH), :])
      if g == 0:
        pltpu.sync_copy(exb, den_sh.at[pl.ds(r0 + bi * CH, CH)])
    plsc.subcore_barrier()

    _issue_gather(0, 0, goff)
    _issue_gather(1, 1, goff)

    def quad_body(i, _):
      for t in range(NBUF):
        j = i * NBUF + t
        bn2 = (t + 2) % NBUF
        @pl.when(j >= 2)
        def _():
          pltpu.make_async_copy(rows.at[bn2], u_sh.at[didx2.at[j - 2]],
                                semsc[bn2]).wait()
        @pl.when(j + 2 < nch)
        def _():
          _issue_gather(j + 2, bn2, goff)

        pltpu.make_async_copy(h_hbm.at[gidx.at[t]], rows.at[t],
                              semg[t]).wait()
        bj = jnp.broadcast_to(j, (16,)).astype(jnp.int32)
        @plsc.parallel_loop(0, CH, 1, unroll=8)
        def _(k):
          bk = jnp.broadcast_to(k, (16,)).astype(jnp.int32)
          sv = plsc.load_gather(exall, [bj, bk])
          for v in range(nvec):
            sl = pl.ds(v * 16, 16)
            rows[t, k, sl] = rows[t, k, sl] * sv

        if g == 0:
          @pl.when(j >= 2)
          def _():
            pltpu.make_async_copy(exall.at[j - 2], den_sh.at[didx2.at[j - 2]],
                                  semd[t % 2]).wait()
        pltpu.async_copy(rows.at[t], u_sh.at[didx2.at[j]], semsc[t], add=True)
        if g == 0:
          pltpu.async_copy(exall.at[j], den_sh.at[didx2.at[j]], semd[t % 2],
                           add=True)
      return 0

    lax.fori_loop(0, nch // NBUF, quad_body, 0)
    for j in (nch - 2, nch - 1):
      bb = j % NBUF
      pltpu.make_async_copy(rows.at[bb], u_sh.at[didx2.at[j]],
                            semsc[bb]).wait()
      if g == 0:
        pltpu.make_async_copy(exall.at[j], den_sh.at[didx2.at[j]],
                              semd[j % 2]).wait()
    plsc.subcore_barrier()

    for bi in range(ROWS_PER_TILE // CH):
      rsl = pl.ds(r0 + bi * CH, CH)
      pltpu.sync_copy(u_sh.at[rsl, :], rows.at[0])
      if div:
        pltpu.sync_copy(den_sh.at[rsl], exb)
        def rdi(i, _):
          sl = pl.ds(i * 16, 16)
          d = exb[sl]
          exb[sl] = 1.0 / jnp.where(d > 0, d, 1.0)
          return 0
        lax.fori_loop(0, CH // 16, rdi, 0)

        @plsc.parallel_loop(0, CH, 1, unroll=4)
        def _(k):
          bk = jnp.broadcast_to(k, (16,)).astype(jnp.int32)
          rv = plsc.load_gather(exb, [bk])
          for v in range(nvec):
            sl = pl.ds(v * 16, 16)
            rows[0, k, sl] = rows[0, k, sl] * rv
        pltpu.sync_copy(rows.at[0], x2_hbm.at[grp, rsl, :])
      else:
        pltpu.sync_copy(rows.at[0], u_hbm.at[c, rsl, :])
        pltpu.sync_copy(den_sh.at[rsl], exb)
        pltpu.sync_copy(exb, den_hbm.at[c, rsl])


def _sc_layer(h, srcm, dstm, esedt, *, d2, ngrp, fsplit, div):
  nch = (EPAD // CH) // (NS if fsplit else (NS * NC))
  mesh = plsc.VectorSubcoreMesh(core_axis_name="c", subcore_axis_name="s",
                                num_cores=NC, num_subcores=NS)
  if div:
    out_type = jax.ShapeDtypeStruct((NC * ngrp, NPAD, d2), f32)
  else:
    out_type = (jax.ShapeDtypeStruct((2, NPAD, d2), f32),
                jax.ShapeDtypeStruct((2, NPAD), f32))
  scratch = [
      pltpu.VMEM((NPAD,), f32),
      pltpu.VMEM((NPAD,), f32),
      pltpu.VMEM((nch, CH), jnp.int32),
      pltpu.VMEM((nch, CH), jnp.int32),
      pltpu.VMEM((NBUF, CH), jnp.int32),
      pltpu.VMEM((NBUF, CH, d2), f32),
      pltpu.VMEM((nch, CH), f32),
      pltpu.VMEM((CH,), f32),
      pltpu.VMEM_SHARED((NPAD, d2), f32),
      pltpu.VMEM_SHARED((NPAD,), f32),
  ] + [pltpu.SemaphoreType.DMA] * (2 * NBUF + 2)

  def body(*refs):
    _sc_edge_body(refs, d2=d2, ngrp=ngrp, fsplit=fsplit, div=div)

  cp = pltpu.CompilerParams(
      needs_layout_passes=False,
      use_tc_tiling_on_sc=False)
  fn = pl.kernel(body, out_type=out_type, mesh=mesh, scratch_types=scratch,
                 compiler_params=cp)
  return fn(h, srcm, dstm, esedt)


def kernel(x, edge_index, W1, a_src1, a_dst1, b1, W2, a_src2, a_dst2, b2,
           W3, a_src3, a_dst3, b3):
  src = edge_index[0]
  dst = edge_index[1]
  npad_e = EPAD - E
  padidx = (jnp.arange(npad_e, dtype=jnp.int32) % (NPAD - N)) + N
  srcm = jnp.concatenate([src, padidx]).reshape(EPAD // CH, CH)
  dstm = jnp.concatenate([dst, padidx]).reshape(EPAD // CH, CH)
  xp = jnp.pad(x, ((0, NPAD - N), (0, 0)))

  bc8 = lambda v: jnp.broadcast_to(v[None, :], (8, v.shape[0]))
  w3p = jnp.pad(W3, ((0, 0), (0, 8)))
  a_src3p = jnp.pad(a_src3, (0, 8))
  a_dst3p = jnp.pad(a_dst3, (0, 8))
  b3p = jnp.pad(b3, (0, 8))

  L1G = 2
  ngroups = NC * L1G
  dq = 128 // ngroups
  esed1, x1g = _tc_esed1(xp, W1, bc8(a_src1), bc8(a_dst1), ngroups)
  v1 = _sc_layer(x1g.reshape(ngroups * NPAD, dq), srcm, dstm, esed1.T,
                 d2=dq, ngrp=L1G, fsplit=True, div=True)

  h2, esed2 = _tc_l2(v1, W1, bc8(b1), W2, bc8(a_src2), bc8(a_dst2))
  u2, den2 = _sc_layer(h2, srcm, dstm, esed2.T,
                       d2=32, ngrp=1, fsplit=False, div=False)

  x3, esed3 = _tc_l3(u2[0], u2[1], den2[0][:, None], den2[1][:, None],
                     bc8(b2), w3p, bc8(a_src3p), bc8(a_dst3p))
  v3, den3 = _sc_layer(x3, srcm, dstm, esed3.T,
                       d2=32, ngrp=1, fsplit=False, div=False)

  return _tc_final(v3[0], v3[1], den3[0][:, None], den3[1][:, None],
                   bc8(b3p), w3p)
